# baseline (speedup 1.0000x reference)
import functools
import math

import jax
import jax.numpy as jnp
from jax import lax
from jax.experimental import pallas as pl
from jax.experimental.pallas import tpu as pltpu
from jax.experimental.pallas import tpu_sc as plsc

_SC_MESH = dict(mesh=plsc.VectorSubcoreMesh(core_axis_name="c",
                                            subcore_axis_name="s"),
                compiler_params=pltpu.CompilerParams(
                    needs_layout_passes=False))
_NW = 32

_N = 10000
_E = 320000
_H = 128
_NP = 10240
_K1, _K1P = 5000, 5120
_K2, _K2P = 2500, 2560
_K3, _K3P = 1250, 1280

_INT_MIN = -2147483648


def _bs(n):
    return 512 if n % 512 == 0 else 256


def _feat_mm_body(x_ref, w_ref, s_ref, b_ref, o_ref, *, act, use_scale):
    acc = jnp.dot(x_ref[...], w_ref[...], preferred_element_type=jnp.float32)
    if use_scale:
        acc = acc * s_ref[...]
    acc = acc + b_ref[...]
    if act:
        acc = jnp.where(acc > 0, acc, jnp.exp(acc) - 1.0)
    o_ref[...] = acc


def _feat_mm(x, W, b, scale=None, act=False, bm=None):
    n = x.shape[0]
    bm = bm or _bs(n)
    use_scale = scale is not None
    if scale is None:
        scale = jnp.zeros((n, 1), jnp.float32)
    grid = (n // bm,)
    return pl.pallas_call(
        functools.partial(_feat_mm_body, act=act, use_scale=use_scale),
        grid=grid,
        in_specs=[
            pl.BlockSpec((bm, _H), lambda i: (i, 0)),
            pl.BlockSpec((_H, _H), lambda i: (0, 0)),
            pl.BlockSpec((bm, 1), lambda i: (i, 0)),
            pl.BlockSpec((1, _H), lambda i: (0, 0)),
        ],
        out_specs=pl.BlockSpec((bm, _H), lambda i: (i, 0)),
        out_shape=jax.ShapeDtypeStruct((n, _H), jnp.float32),
    )(x, W, scale, b.reshape(1, _H))


def _colsum_body(m_ref, o_ref):
    i = pl.program_id(1)

    @pl.when(i == 0)
    def _():
        o_ref[...] = jnp.zeros_like(o_ref)

    o_ref[...] += jnp.sum(m_ref[...], axis=0, keepdims=True)


def _colsum(M, bi=None, bj=None):
    n, m = M.shape
    bi = bi or _bs(n)
    bj = bj or _bs(m)
    return pl.pallas_call(
        _colsum_body,
        grid=(m // bj, n // bi),
        in_specs=[pl.BlockSpec((bi, bj), lambda j, i: (i, j))],
        out_specs=pl.BlockSpec((1, bj), lambda j, i: (0, j)),
        out_shape=jax.ShapeDtypeStruct((1, m), jnp.float32),
    )(M)


def _agg_body(m_ref, y_ref, y2_ref, d_ref, b_ref, o_ref, acc, *, nk, act):
    k = pl.program_id(1)

    @pl.when(k == 0)
    def _():
        acc[...] = jnp.zeros_like(acc)

    acc[...] += jax.lax.dot_general(
        m_ref[...], y_ref[...], (((0,), (0,)), ((), ())),
        preferred_element_type=jnp.float32)

    @pl.when(k == nk - 1)
    def _():
        r = (acc[...] + 2.0 * y2_ref[...]) * d_ref[...] + b_ref[...]
        if act:
            r = jnp.where(r > 0, r, jnp.exp(r) - 1.0)
        o_ref[...] = r


def _gcn_agg(M, y, dinv_col, b, act, bi=None, bk=None):
    n = M.shape[0]
    bi = bi or _bs(n)
    bk = bk or _bs(n)
    return pl.pallas_call(
        functools.partial(_agg_body, nk=n // bk, act=act),
        grid=(n // bi, n // bk),
        in_specs=[
            pl.BlockSpec((bk, bi), lambda i, k: (k, i)),
            pl.BlockSpec((bk, _H), lambda i, k: (k, 0)),
            pl.BlockSpec((bi, _H), lambda i, k: (i, 0)),
            pl.BlockSpec((bi, 1), lambda i, k: (i, 0)),
            pl.BlockSpec((1, _H), lambda i, k: (0, 0)),
        ],
        out_specs=pl.BlockSpec((bi, _H), lambda i, k: (i, 0)),
        out_shape=jax.ShapeDtypeStruct((n, _H), jnp.float32),
        scratch_shapes=[pltpu.VMEM((bi, _H), jnp.float32)],
    )(M, y, y, dinv_col, b.reshape(1, _H))


def _gcn_dense(M, x, W, b, act):
    deg = _colsum(M) + 2.0
    dinv_col = jax.lax.rsqrt(deg).reshape(-1, 1)
    y = _feat_mm(x, W, jnp.zeros((_H,), jnp.float32), scale=dinv_col)
    return _gcn_agg(M, y, dinv_col, b, act)


def _bigmm_body(p_ref, q_ref, d_ref, o_ref, acc, *, nk, use_d, diag_zero, bm, bn):
    mi = pl.program_id(0)
    nj = pl.program_id(1)
    k = pl.program_id(2)

    @pl.when(k == 0)
    def _():
        acc[...] = jnp.zeros_like(acc)

    acc[...] += jnp.dot(p_ref[...].astype(jnp.bfloat16),
                        q_ref[...].astype(jnp.bfloat16),
                        preferred_element_type=jnp.float32)

    @pl.when(k == nk - 1)
    def _():
        r = acc[...]
        if use_d:
            r = r + 2.0 * d_ref[...]
        if diag_zero:
            rows = mi * bm + jax.lax.broadcasted_iota(jnp.int32, (bm, bn), 0)
            cols = nj * bn + jax.lax.broadcasted_iota(jnp.int32, (bm, bn), 1)
            r = jnp.where(rows == cols, 0.0, r)
        o_ref[...] = r


def _big_mm(P, Q, D=None, diag_zero=False, bm=None, bn=None, bk=None):
    m, kdim = P.shape
    n = Q.shape[1]
    bm = bm or _bs(m)
    bn = bn or _bs(n)
    bk = bk or _bs(kdim)
    use_d = D is not None
    if D is None:
        D = jnp.zeros((bm, bn), jnp.float32)
        d_spec = pl.BlockSpec((bm, bn), lambda i, j, k: (0, 0))
    else:
        d_spec = pl.BlockSpec((bm, bn), lambda i, j, k: (i, j))
    return pl.pallas_call(
        functools.partial(_bigmm_body, nk=kdim // bk, use_d=use_d,
                          diag_zero=diag_zero, bm=bm, bn=bn),
        grid=(m // bm, n // bn, kdim // bk),
        in_specs=[
            pl.BlockSpec((bm, bk), lambda i, j, k: (i, k)),
            pl.BlockSpec((bk, bn), lambda i, j, k: (k, j)),
            d_spec,
        ],
        out_specs=pl.BlockSpec((bm, bn), lambda i, j, k: (i, j)),
        out_shape=jax.ShapeDtypeStruct((m, n), jnp.float32),
        scratch_shapes=[pltpu.VMEM((bm, bn), jnp.float32)],
    )(P, Q, D)


def _score_body(p_ref, h_ref, o_ref):
    pr = p_ref[...]
    inv_norm = jax.lax.rsqrt(jnp.sum(pr * pr))
    s = jax.lax.dot_general(pr, h_ref[...], (((1,), (1,)), ((), ())),
                            preferred_element_type=jnp.float32)
    o_ref[...] = jnp.tanh(s * inv_norm)


def _scores(h, p):
    n = h.shape[0]
    nb = n // _H
    return pl.pallas_call(
        _score_body,
        grid=(nb,),
        in_specs=[
            pl.BlockSpec((1, _H), lambda b: (0, 0)),
            pl.BlockSpec((_H, _H), lambda b: (b, 0)),
        ],
        out_specs=pl.BlockSpec((1, _H), lambda b: (0, b)),
        out_shape=jax.ShapeDtypeStruct((1, n), jnp.float32),
    )(p.reshape(1, _H), h)


def _topk_body(s_ref, ip_ref, *, n_real, k):
    nb = s_ref.shape[0]
    s = s_ref[...]
    rows = jax.lax.broadcasted_iota(jnp.int32, (nb, _H), 0)
    cols = jax.lax.broadcasted_iota(jnp.int32, (nb, _H), 1)
    gidx = rows * _H + cols
    valid = gidx < n_real
    u = jax.lax.bitcast_convert_type(s, jnp.int32)
    key = u ^ jnp.where(u < 0, 0x7FFFFFFF, 0)
    key = jnp.where(valid, key, _INT_MIN)

    def body(b, prefix):
        cand = jnp.where(b == 31, jnp.int32(0), prefix + (jnp.int32(1) << b))
        cnt = jnp.sum(jnp.where(key >= cand, 1.0, 0.0))
        return jnp.where(cnt >= k, cand, prefix)

    thr = jax.lax.fori_loop(0, 32, lambda i, c: body(31 - i, c),
                            jnp.int32(_INT_MIN), unroll=True)

    gt = jnp.where(key > thr, 1.0, 0.0)
    eq = jnp.where(key == thr, 1.0, 0.0)
    cnt_gt = jnp.sum(gt)
    r_need = k - cnt_gt

    upper = jnp.where(
        jax.lax.broadcasted_iota(jnp.int32, (_H, _H), 0)
        < jax.lax.broadcasted_iota(jnp.int32, (_H, _H), 1), 1.0, 0.0)
    lower_nb = jnp.where(
        jax.lax.broadcasted_iota(jnp.int32, (nb, nb), 1)
        < jax.lax.broadcasted_iota(jnp.int32, (nb, nb), 0), 1.0, 0.0)
    ones_col = jnp.ones((_H, 1), jnp.float32)

    def ex_prefix(m):
        within = jnp.dot(m, upper, preferred_element_type=jnp.float32)
        rowsum = jnp.dot(m, ones_col, preferred_element_type=jnp.float32)
        rowoff = jnp.dot(lower_nb, rowsum, preferred_element_type=jnp.float32)
        return within + rowoff

    eq_rank = ex_prefix(eq)
    sel = (gt > 0) | ((eq > 0) & (eq_rank < r_need))
    self32 = jnp.where(sel, 1.0, 0.0)
    slot = ex_prefix(self32)
    ip_ref[...] = jnp.where(sel, slot.astype(jnp.int32), -1)


def _topk_ip(s2d, n_real, k):
    nb = s2d.shape[0]
    return pl.pallas_call(
        functools.partial(_topk_body, n_real=n_real, k=k),
        in_specs=[pl.BlockSpec((nb, _H), lambda: (0, 0))],
        out_specs=pl.BlockSpec((nb, _H), lambda: (0, 0)),
        out_shape=jax.ShapeDtypeStruct((nb, _H), jnp.int32),
    )(s2d)


def _readout_body(u_ref, w_ref, b_ref, o_ref, acc, *, n_real, nb, bm):
    i = pl.program_id(0)

    @pl.when(i == 0)
    def _():
        acc[...] = jnp.zeros_like(acc)

    rows = i * bm + jax.lax.broadcasted_iota(jnp.int32, (bm, _H), 0)
    blk = jnp.where(rows < n_real, u_ref[...], 0.0)
    acc[...] += jnp.sum(blk, axis=0, keepdims=True)

    @pl.when(i == nb - 1)
    def _():
        g = acc[...] * (1.0 / n_real)
        o_ref[...] = jnp.dot(g, w_ref[...],
                             preferred_element_type=jnp.float32) + b_ref[...]


def _readout(u, Wo, bo, n_real, bm=512):
    n = u.shape[0]
    nb = n // bm
    return pl.pallas_call(
        functools.partial(_readout_body, n_real=n_real, nb=nb, bm=bm),
        grid=(nb,),
        in_specs=[
            pl.BlockSpec((bm, _H), lambda i: (i, 0)),
            pl.BlockSpec((_H, _H), lambda i: (0, 0)),
            pl.BlockSpec((1, _H), lambda i: (0, 0)),
        ],
        out_specs=pl.BlockSpec((1, _H), lambda i: (0, 0)),
        out_shape=jax.ShapeDtypeStruct((1, _H), jnp.float32),
        scratch_shapes=[pltpu.VMEM((1, _H), jnp.float32)],
    )(u, Wo, bo.reshape(1, _H))


def _sc_wid():
    return lax.axis_index("c") * 16 + lax.axis_index("s")


def _fill_1d(ref, n, val, dtype):
    for i in range(n // 16):
        ref[pl.ds(i * 16, 16)] = jnp.full((16,), val, dtype)


def _sc_deg(dst):
    ept = _E // _NW
    nwin = ept // 80
    rows = _NP // 16

    @functools.partial(
        pl.kernel,
        out_type=jax.ShapeDtypeStruct((2, _NP), jnp.float32),
        scratch_types=[
            pltpu.VMEM((80,), jnp.int32),
            pltpu.VMEM((80,), jnp.float32),
            pltpu.VMEM((rows,), jnp.float32),
            pltpu.VMEM_SHARED((_NP,), jnp.float32),
        ],
        **_SC_MESH,
    )
    def k(dst_hbm, out_hbm, idxw, ones_v, zbuf, degsh):
        cid = lax.axis_index("c")
        sid = lax.axis_index("s")
        _fill_1d(ones_v, 80, 1.0, jnp.float32)
        _fill_1d(zbuf, rows, 0.0, jnp.float32)
        pltpu.sync_copy(zbuf, degsh.at[pl.ds(sid * rows, rows)])
        plsc.subcore_barrier()
        base = (cid * 16 + sid) * ept

        def win(w, c):
            pltpu.sync_copy(dst_hbm.at[pl.ds(base + w * 80, 80)], idxw)
            pltpu.sync_copy(ones_v, degsh.at[idxw], add=True)
            return c

        lax.fori_loop(0, nwin, win, 0)
        plsc.subcore_barrier()
        pltpu.sync_copy(degsh.at[pl.ds(sid * rows, rows)],
                        out_hbm.at[cid, pl.ds(sid * rows, rows)])

    return k(dst)


def _sc_agg(y, src, dst):
    ept = _E // _NW
    nwin = ept // 80
    rows = _NP // 16

    @functools.partial(
        pl.kernel,
        out_type=jax.ShapeDtypeStruct((2, _NP, _H), jnp.float32),
        scratch_types=[
            pltpu.VMEM((80,), jnp.int32),
            pltpu.VMEM((80,), jnp.int32),
            pltpu.VMEM((80,), jnp.int32),
            pltpu.VMEM((80,), jnp.int32),
            pltpu.VMEM((80, _H), jnp.float32),
            pltpu.VMEM((80, _H), jnp.float32),
            pltpu.VMEM((16, _H), jnp.float32),
            pltpu.VMEM_SHARED((_NP, _H), jnp.float32),
            pltpu.SemaphoreType.DMA,
            pltpu.SemaphoreType.DMA,
            pltpu.SemaphoreType.DMA,
            pltpu.SemaphoreType.DMA,
        ],
        **_SC_MESH,
    )
    def k(y_hbm, src_hbm, dst_hbm, out_hbm, sidx, didx, sidx1, didx1,
          rbuf, rbuf1, zbuf, accsh, sem0, sem1, sem2, sem3):
        cid = lax.axis_index("c")
        sid = lax.axis_index("s")
        for r in range(16):
            _fill_1d(zbuf.at[r], _H, 0.0, jnp.float32)

        def zro(r, c):
            pltpu.sync_copy(zbuf, accsh.at[pl.ds(sid * rows + r * 16, 16)])
            return c

        lax.fori_loop(0, rows // 16, zro, 0)
        plsc.subcore_barrier()
        base = (cid * 16 + sid) * ept

        def win(w, c):
            pltpu.sync_copy(src_hbm.at[pl.ds(base + 2 * w * 80, 80)], sidx)
            pltpu.sync_copy(dst_hbm.at[pl.ds(base + 2 * w * 80, 80)], didx)
            g0 = pltpu.async_copy(y_hbm.at[sidx], rbuf, sem0)
            pltpu.sync_copy(src_hbm.at[pl.ds(base + (2 * w + 1) * 80, 80)],
                            sidx1)
            pltpu.sync_copy(dst_hbm.at[pl.ds(base + (2 * w + 1) * 80, 80)],
                            didx1)
            g1 = pltpu.async_copy(y_hbm.at[sidx1], rbuf1, sem1)
            g0.wait()
            s0 = pltpu.async_copy(rbuf, accsh.at[didx], sem2, add=True)
            g1.wait()
            s1 = pltpu.async_copy(rbuf1, accsh.at[didx1], sem3, add=True)
            s0.wait()
            s1.wait()
            return c

        lax.fori_loop(0, nwin // 2, win, 0)
        pltpu.sync_copy(src_hbm.at[pl.ds(base + (nwin - 1) * 80, 80)], sidx)
        pltpu.sync_copy(dst_hbm.at[pl.ds(base + (nwin - 1) * 80, 80)], didx)
        pltpu.async_copy(y_hbm.at[sidx], rbuf, sem0).wait()
        pltpu.sync_copy(rbuf, accsh.at[didx], add=True)
        plsc.subcore_barrier()
        pltpu.sync_copy(accsh.at[pl.ds(sid * rows, rows)],
                        out_hbm.at[cid, pl.ds(sid * rows, rows)])

    return k(y, src, dst)


def _sc_compact(ip_flat, s_flat, n, kp, zero_row):
    nwin = n // 2048

    @functools.partial(
        pl.kernel,
        out_type=(jax.ShapeDtypeStruct((kp,), jnp.int32),
                  jax.ShapeDtypeStruct((kp,), jnp.float32)),
        scratch_types=[
            pltpu.VMEM((2048,), jnp.int32),
            pltpu.VMEM((2048,), jnp.float32),
            pltpu.VMEM((kp,), jnp.int32),
            pltpu.VMEM((kp,), jnp.float32),
        ],
        **_SC_MESH,
    )
    def k(ip_hbm, s_hbm, perm_hbm, vals_hbm, ipw, sw, permb, valsb):
        wid = _sc_wid()

        @pl.when(wid == 0)
        def _():
            _fill_1d(permb, kp, zero_row, jnp.int32)
            _fill_1d(valsb, kp, 0.0, jnp.float32)

            def win(w, c):
                pltpu.sync_copy(ip_hbm.at[pl.ds(w * 2048, 2048)], ipw)
                pltpu.sync_copy(s_hbm.at[pl.ds(w * 2048, 2048)], sw)

                def inner(i, c2):
                    idx = ipw[pl.ds(i * 16, 16)]
                    msk = idx >= 0
                    idx2 = jnp.maximum(idx, 0)
                    gi = (w * 2048 + i * 16
                          + lax.iota(jnp.int32, 16))
                    plsc.store_scatter(permb, [idx2], gi, mask=msk)
                    plsc.store_scatter(valsb, [idx2], sw[pl.ds(i * 16, 16)],
                                       mask=msk)
                    return c2

                lax.fori_loop(0, 128, inner, 0)
                return c

            lax.fori_loop(0, nwin, win, 0)
            pltpu.sync_copy(permb, perm_hbm)
            pltpu.sync_copy(valsb, vals_hbm)

    return k(ip_flat, s_flat)


def _sc_rowgather(T, idx, clamp=False):
    kp = idx.shape[0]
    w = T.shape[1]
    rpw = kp // _NW
    nwin = rpw // 16

    @functools.partial(
        pl.kernel,
        out_type=jax.ShapeDtypeStruct((kp, w), T.dtype),
        scratch_types=[
            pltpu.VMEM((16,), jnp.int32),
            pltpu.VMEM((16, w), T.dtype),
            pltpu.SemaphoreType.DMA,
        ],
        **_SC_MESH,
    )
    def k(t_hbm, idx_hbm, out_hbm, idxw, rbuf, sem):
        base = _sc_wid() * rpw

        def win(r, c):
            pltpu.sync_copy(idx_hbm.at[pl.ds(base + r * 16, 16)], idxw)
            if clamp:
                idxw[pl.ds(0, 16)] = jnp.maximum(idxw[pl.ds(0, 16)], 0)
            pltpu.async_copy(t_hbm.at[idxw], rbuf, sem).wait()
            pltpu.sync_copy(rbuf, out_hbm.at[pl.ds(base + r * 16, 16)])
            return c

        lax.fori_loop(0, nwin, win, 0)

    return k(T, idx)


def _sc_colgather(T, idx):
    m, w = T.shape
    kp = idx.shape[0]
    rpw = m // _NW

    @functools.partial(
        pl.kernel,
        out_type=jax.ShapeDtypeStruct((m, kp), jnp.float32),
        scratch_types=[
            pltpu.VMEM((kp,), jnp.int32),
            pltpu.VMEM((w,), jnp.float32),
            pltpu.VMEM((kp,), jnp.float32),
        ],
        **_SC_MESH,
    )
    def k(t_hbm, idx_hbm, out_hbm, idxall, rowb, outb):
        base = _sc_wid() * rpw
        pltpu.sync_copy(idx_hbm, idxall)

        def row(r, c):
            pltpu.sync_copy(t_hbm.at[base + r], rowb)

            def inner(j, c2):
                iv = idxall[pl.ds(j * 16, 16)]
                outb[pl.ds(j * 16, 16)] = plsc.load_gather(rowb, [iv])
                return c2

            lax.fori_loop(0, kp // 16, inner, 0)
            pltpu.sync_copy(outb, out_hbm.at[base + r])
            return c

        lax.fori_loop(0, rpw, row, 0)

    return k(T, idx)


_CHUNK = 819200
_EPT3 = _E // _NW
_EPT3P = 10240


def _sc_build_bcd(src, dst, ip):
    nwin = _EPT3P // 128

    @functools.partial(
        pl.kernel,
        out_type=(jax.ShapeDtypeStruct((_NP * _K1P,), jnp.float32),
                  jax.ShapeDtypeStruct((_K1P * _NP,), jnp.float32),
                  jax.ShapeDtypeStruct((_K1P * _K1P,), jnp.float32)),
        scratch_types=[
            pltpu.VMEM((_EPT3P,), jnp.int32),
            pltpu.VMEM((_EPT3P,), jnp.int32),
            pltpu.VMEM((_EPT3P,), jnp.int32),
            pltpu.VMEM((_EPT3P,), jnp.int32),
            pltpu.VMEM((_NP,), jnp.int32),
            pltpu.VMEM((10240,), jnp.float32),
            pltpu.VMEM((128,), jnp.int32),
            pltpu.VMEM((128,), jnp.int32),
            pltpu.VMEM((128,), jnp.float32),
            pltpu.VMEM_SHARED((_CHUNK,), jnp.float32),
            pltpu.SemaphoreType.DMA,
            pltpu.SemaphoreType.DMA,
        ],
        **_SC_MESH,
    )
    def k(src_hbm, dst_hbm, ip_hbm, b_hbm, c_hbm, d_hbm,
          srcc, dstc, ipsc, ipdc, iptab, zbuf, idxw, idxw1, ones_v, chunk,
          sem0, sem1):
        cid = lax.axis_index("c")
        sid = lax.axis_index("s")
        ebase = (cid * 16 + sid) * _EPT3
        pltpu.sync_copy(src_hbm.at[pl.ds(ebase, _EPT3)],
                        srcc.at[pl.ds(0, _EPT3)])
        pltpu.sync_copy(dst_hbm.at[pl.ds(ebase, _EPT3)],
                        dstc.at[pl.ds(0, _EPT3)])
        pltpu.sync_copy(ip_hbm, iptab)

        def fill(i, c):
            zbuf[pl.ds(i * 16, 16)] = jnp.zeros((16,), jnp.float32)
            return c

        lax.fori_loop(0, 10240 // 16, fill, 0)
        _fill_1d(ones_v, 128, 1.0, jnp.float32)

        def pre(i, c):
            gi = i * 16 + lax.iota(jnp.int32, 16)
            inb = gi < _EPT3
            s = jnp.where(inb, srcc[pl.ds(i * 16, 16)], 0)
            d = jnp.where(inb, dstc[pl.ds(i * 16, 16)], 0)
            ok = inb & (s != d)
            ipsc[pl.ds(i * 16, 16)] = jnp.where(
                ok, plsc.load_gather(iptab, [s]), -1)
            ipdc[pl.ds(i * 16, 16)] = jnp.where(
                ok, plsc.load_gather(iptab, [d]), -1)
            return c

        lax.fori_loop(0, _EPT3P // 16, pre, 0)

        tel = sid * 51200

        def phase(rowarr, colarr, out_hbm, rows, width, nchunks):
            def one_chunk(j, c):
                r0 = (cid * nchunks + j) * rows

                def zro(z, c2):
                    pltpu.sync_copy(
                        zbuf, chunk.at[pl.ds(tel + z * 10240, 10240)])
                    return c2

                lax.fori_loop(0, 5, zro, 0)
                plsc.subcore_barrier()

                rpt = rows // 16
                drow = sid * rpt + lax.rem(lax.iota(jnp.int32, 16), rpt)
                dump = drow * width + (width - 1)

                def build(w, buf):
                    for v in range(8):
                        sl = pl.ds(w * 128 + v * 16, 16)
                        rv = rowarr[sl]
                        cv = colarr[sl]
                        ok = (rv >= r0) & (rv < r0 + rows) & (cv >= 0)
                        lidx = (rv - r0) * width + cv
                        buf[pl.ds(v * 16, 16)] = jnp.where(ok, lidx, dump)

                def win(w, c2):
                    build(2 * w, idxw)
                    cp0 = pltpu.async_copy(ones_v, chunk.at[idxw], sem0,
                                           add=True)
                    build(2 * w + 1, idxw1)
                    cp1 = pltpu.async_copy(ones_v, chunk.at[idxw1], sem1,
                                           add=True)
                    cp0.wait()
                    cp1.wait()
                    return c2

                lax.fori_loop(0, nwin // 2, win, 0)
                plsc.subcore_barrier()
                pltpu.sync_copy(
                    chunk.at[pl.ds(tel, 51200)],
                    out_hbm.at[pl.ds(r0 * width + tel, 51200)])
                plsc.subcore_barrier()
                return c

            lax.fori_loop(0, nchunks, one_chunk, 0)

        phase(srcc, ipdc, b_hbm, 160, _K1P, 32)
        phase(ipsc, dstc, c_hbm, 80, _NP, 32)
        phase(ipsc, ipdc, d_hbm, 160, _K1P, 16)

    return k(src, dst, ip)


def _ew_gcn0_body(p0_ref, p1_ref, y_ref, d_ref, b_ref, o_ref, *, act):
    r = (p0_ref[...] + p1_ref[...] + 2.0 * y_ref[...]) * d_ref[...] + b_ref[...]
    if act:
        r = jnp.where(r > 0, r, jnp.exp(r) - 1.0)
    o_ref[...] = r


def _ew_gcn0(P0, P1, y, dinv_col, b, act, bm=512):
    n = y.shape[0]
    return pl.pallas_call(
        functools.partial(_ew_gcn0_body, act=act),
        grid=(n // bm,),
        in_specs=[pl.BlockSpec((bm, _H), lambda i: (i, 0))] * 3
        + [pl.BlockSpec((bm, 1), lambda i: (i, 0)),
           pl.BlockSpec((1, _H), lambda i: (0, 0))],
        out_specs=pl.BlockSpec((bm, _H), lambda i: (i, 0)),
        out_shape=jax.ShapeDtypeStruct((n, _H), jnp.float32),
    )(P0, P1, y, dinv_col, b.reshape(1, _H))


def _mask_add_body(res_ref, g_ref, ip_ref, o_ref):
    o_ref[...] = res_ref[...] + jnp.where(ip_ref[...] >= 0, g_ref[...], 0.0)


def _mask_add(res, g, ip_col, bm=None):
    n = res.shape[0]
    bm = bm or _bs(n)
    return pl.pallas_call(
        _mask_add_body,
        grid=(n // bm,),
        in_specs=[pl.BlockSpec((bm, _H), lambda i: (i, 0)),
                  pl.BlockSpec((bm, _H), lambda i: (i, 0)),
                  pl.BlockSpec((bm, 1), lambda i: (i, 0))],
        out_specs=pl.BlockSpec((bm, _H), lambda i: (i, 0)),
        out_shape=jax.ShapeDtypeStruct((n, _H), jnp.float32),
    )(res, g, ip_col)


def _adj_comb_body(a_ref, d_ref, o_ref, *, bm, bn):
    i = pl.program_id(0)
    j = pl.program_id(1)
    r = a_ref[...] + 2.0 * d_ref[...]
    rows = i * bm + jax.lax.broadcasted_iota(jnp.int32, (bm, bn), 0)
    cols = j * bn + jax.lax.broadcasted_iota(jnp.int32, (bm, bn), 1)
    o_ref[...] = jnp.where(rows == cols, 0.0, r)


def _adj_combine(A, D, bm=None, bn=None):
    m, n = A.shape
    bm = bm or _bs(m)
    bn = bn or _bs(n)
    return pl.pallas_call(
        functools.partial(_adj_comb_body, bm=bm, bn=bn),
        grid=(m // bm, n // bn),
        in_specs=[pl.BlockSpec((bm, bn), lambda i, j: (i, j)),
                  pl.BlockSpec((bm, bn), lambda i, j: (i, j))],
        out_specs=pl.BlockSpec((bm, bn), lambda i, j: (i, j)),
        out_shape=jax.ShapeDtypeStruct((m, n), jnp.float32),
    )(A, D)


def _topk_stage(h, p, n_real, k, kp):
    s = _scores(h, p)
    np_ = h.shape[0]
    ip2d = _topk_ip(s.reshape(-1, _H), n_real, k)
    ip = ip2d.reshape(-1)
    perm, vals = _sc_compact(ip, s.reshape(-1), np_, kp, zero_row=n_real)
    return perm, vals, ip


def _restricted_square(M, perm):
    C = _sc_rowgather(M, perm)
    D = _sc_colgather(C, perm)
    G = _big_mm(C, M)
    Gc = _sc_colgather(G, perm)
    return _adj_combine(Gc, D)


def _gcn_dense(M, x, W, b, act, extra=None):
    deg = _colsum(M) + 2.0
    dinv_col = jax.lax.rsqrt(deg).reshape(-1, 1)
    scale = dinv_col if extra is None else dinv_col * extra
    y = _feat_mm(x, W, jnp.zeros((_H,), jnp.float32), scale=scale)
    return _gcn_agg(M, y, dinv_col, b, act)


def kernel(x, edge_index, batch, Wd0, bd0, Wd1, bd1, Wd2, bd2, Wd3, bd3,
           p1, p2, p3, Wu0, bu0, Wu1, bu1, Wu2, bu2, Wo, bo):
    src = edge_index[0]
    dst = edge_index[1]
    xp = jnp.pad(x, ((0, _NP - _N), (0, 0)))

    degp = _sc_deg(dst)
    deg0 = degp[0] + degp[1] + 2.0
    dinv0c = jax.lax.rsqrt(deg0).reshape(-1, 1)

    def gcn_edges(xin, W, b, act):
        y = _feat_mm(xin, W, jnp.zeros((_H,), jnp.float32), scale=dinv0c)
        P = _sc_agg(y, src, dst)
        return _ew_gcn0(P[0], P[1], y, dinv0c, b, act)

    h0 = gcn_edges(xp, Wd0, bd0, True)

    perm1, vals1, ip1 = _topk_stage(h0, p1, _N, _K1, _K1P)
    Bf, Cf, Df = _sc_build_bcd(src, dst, ip1)
    M1 = _big_mm(Cf.reshape(_K1P, _NP), Bf.reshape(_NP, _K1P),
                 D=Df.reshape(_K1P, _K1P), diag_zero=True)
    h1 = _gcn_dense(M1, _sc_rowgather(h0, perm1), Wd1, bd1, True,
                    extra=vals1.reshape(-1, 1))

    perm2, vals2, ip2 = _topk_stage(h1, p2, _K1, _K2, _K2P)
    M2 = _restricted_square(M1, perm2)
    h2 = _gcn_dense(M2, _sc_rowgather(h1, perm2), Wd2, bd2, True,
                    extra=vals2.reshape(-1, 1))

    perm3, vals3, ip3 = _topk_stage(h2, p3, _K2, _K3, _K3P)
    M3 = _restricted_square(M2, perm3)
    h3 = _gcn_dense(M3, _sc_rowgather(h2, perm3), Wd3, bd3, True,
                    extra=vals3.reshape(-1, 1))

    u = _mask_add(h2, _sc_rowgather(h3, ip3, clamp=True), ip3.reshape(-1, 1))
    u = _gcn_dense(M2, u, Wu0, bu0, True)
    u = _mask_add(h1, _sc_rowgather(u, ip2, clamp=True), ip2.reshape(-1, 1))
    u = _gcn_dense(M1, u, Wu1, bu1, True)
    u = _mask_add(h0, _sc_rowgather(u, ip1, clamp=True), ip1.reshape(-1, 1))
    u = gcn_edges(u, Wu2, bu2, False)

    return _readout(u, Wo, bo, _N)


# speedup vs baseline: 1.7136x; 1.7136x over previous
import functools
import math

import jax
import jax.numpy as jnp
from jax import lax
from jax.experimental import pallas as pl
from jax.experimental.pallas import tpu as pltpu
from jax.experimental.pallas import tpu_sc as plsc

_SC_MESH = dict(mesh=plsc.VectorSubcoreMesh(core_axis_name="c",
                                            subcore_axis_name="s"),
                compiler_params=pltpu.CompilerParams(
                    needs_layout_passes=False))
_NW = 32

_N = 10000
_E = 320000
_H = 128
_NP = 10240
_K1, _K1P = 5000, 5120
_K2, _K2P = 2500, 2560
_K3, _K3P = 1250, 1280

_INT_MIN = -2147483648


def _bs(n):
    return 512 if n % 512 == 0 else 256


def _feat_mm_body(x_ref, w_ref, s_ref, b_ref, o_ref, *, act, use_scale):
    acc = jnp.dot(x_ref[...], w_ref[...], preferred_element_type=jnp.float32)
    if use_scale:
        acc = acc * s_ref[...]
    acc = acc + b_ref[...]
    if act:
        acc = jnp.where(acc > 0, acc, jnp.exp(acc) - 1.0)
    o_ref[...] = acc


def _feat_mm(x, W, b, scale=None, act=False, bm=None):
    n = x.shape[0]
    bm = bm or _bs(n)
    use_scale = scale is not None
    if scale is None:
        scale = jnp.zeros((n, 1), jnp.float32)
    grid = (n // bm,)
    return pl.pallas_call(
        functools.partial(_feat_mm_body, act=act, use_scale=use_scale),
        grid=grid,
        in_specs=[
            pl.BlockSpec((bm, _H), lambda i: (i, 0)),
            pl.BlockSpec((_H, _H), lambda i: (0, 0)),
            pl.BlockSpec((bm, 1), lambda i: (i, 0)),
            pl.BlockSpec((1, _H), lambda i: (0, 0)),
        ],
        out_specs=pl.BlockSpec((bm, _H), lambda i: (i, 0)),
        out_shape=jax.ShapeDtypeStruct((n, _H), jnp.float32),
    )(x, W, scale, b.reshape(1, _H))


def _colsum_body(m_ref, o_ref):
    i = pl.program_id(1)

    @pl.when(i == 0)
    def _():
        o_ref[...] = jnp.zeros_like(o_ref)

    o_ref[...] += jnp.sum(m_ref[...], axis=0, keepdims=True)


def _colsum(M, bi=None, bj=None):
    n, m = M.shape
    bi = bi or _bs(n)
    bj = bj or _bs(m)
    return pl.pallas_call(
        _colsum_body,
        grid=(m // bj, n // bi),
        in_specs=[pl.BlockSpec((bi, bj), lambda j, i: (i, j))],
        out_specs=pl.BlockSpec((1, bj), lambda j, i: (0, j)),
        out_shape=jax.ShapeDtypeStruct((1, m), jnp.float32),
    )(M)


def _agg_body(m_ref, y_ref, y2_ref, d_ref, b_ref, o_ref, acc, *, nk, act):
    k = pl.program_id(1)

    @pl.when(k == 0)
    def _():
        acc[...] = jnp.zeros_like(acc)

    acc[...] += jax.lax.dot_general(
        m_ref[...], y_ref[...], (((0,), (0,)), ((), ())),
        preferred_element_type=jnp.float32)

    @pl.when(k == nk - 1)
    def _():
        r = (acc[...] + 2.0 * y2_ref[...]) * d_ref[...] + b_ref[...]
        if act:
            r = jnp.where(r > 0, r, jnp.exp(r) - 1.0)
        o_ref[...] = r


def _gcn_agg(M, y, dinv_col, b, act, bi=None, bk=None):
    n = M.shape[0]
    bi = bi or _bs(n)
    bk = bk or _bs(n)
    return pl.pallas_call(
        functools.partial(_agg_body, nk=n // bk, act=act),
        grid=(n // bi, n // bk),
        in_specs=[
            pl.BlockSpec((bk, bi), lambda i, k: (k, i)),
            pl.BlockSpec((bk, _H), lambda i, k: (k, 0)),
            pl.BlockSpec((bi, _H), lambda i, k: (i, 0)),
            pl.BlockSpec((bi, 1), lambda i, k: (i, 0)),
            pl.BlockSpec((1, _H), lambda i, k: (0, 0)),
        ],
        out_specs=pl.BlockSpec((bi, _H), lambda i, k: (i, 0)),
        out_shape=jax.ShapeDtypeStruct((n, _H), jnp.float32),
        scratch_shapes=[pltpu.VMEM((bi, _H), jnp.float32)],
    )(M, y, y, dinv_col, b.reshape(1, _H))


def _gcn_dense(M, x, W, b, act):
    deg = _colsum(M) + 2.0
    dinv_col = jax.lax.rsqrt(deg).reshape(-1, 1)
    y = _feat_mm(x, W, jnp.zeros((_H,), jnp.float32), scale=dinv_col)
    return _gcn_agg(M, y, dinv_col, b, act)


def _bigmm_body(p_ref, q_ref, d_ref, o_ref, acc, *, nk, use_d, diag_zero, bm, bn):
    mi = pl.program_id(0)
    nj = pl.program_id(1)
    k = pl.program_id(2)

    @pl.when(k == 0)
    def _():
        acc[...] = jnp.zeros_like(acc)

    acc[...] += jnp.dot(p_ref[...].astype(jnp.bfloat16),
                        q_ref[...].astype(jnp.bfloat16),
                        preferred_element_type=jnp.float32)

    @pl.when(k == nk - 1)
    def _():
        r = acc[...]
        if use_d:
            r = r + 2.0 * d_ref[...]
        if diag_zero:
            rows = mi * bm + jax.lax.broadcasted_iota(jnp.int32, (bm, bn), 0)
            cols = nj * bn + jax.lax.broadcasted_iota(jnp.int32, (bm, bn), 1)
            r = jnp.where(rows == cols, 0.0, r)
        o_ref[...] = r


def _big_mm(P, Q, D=None, diag_zero=False, bm=None, bn=None, bk=None):
    m, kdim = P.shape
    n = Q.shape[1]
    bm = bm or _bs(m)
    bn = bn or _bs(n)
    bk = bk or _bs(kdim)
    use_d = D is not None
    if D is None:
        D = jnp.zeros((bm, bn), jnp.float32)
        d_spec = pl.BlockSpec((bm, bn), lambda i, j, k: (0, 0))
    else:
        d_spec = pl.BlockSpec((bm, bn), lambda i, j, k: (i, j))
    return pl.pallas_call(
        functools.partial(_bigmm_body, nk=kdim // bk, use_d=use_d,
                          diag_zero=diag_zero, bm=bm, bn=bn),
        grid=(m // bm, n // bn, kdim // bk),
        in_specs=[
            pl.BlockSpec((bm, bk), lambda i, j, k: (i, k)),
            pl.BlockSpec((bk, bn), lambda i, j, k: (k, j)),
            d_spec,
        ],
        out_specs=pl.BlockSpec((bm, bn), lambda i, j, k: (i, j)),
        out_shape=jax.ShapeDtypeStruct((m, n), jnp.float32),
        scratch_shapes=[pltpu.VMEM((bm, bn), jnp.float32)],
    )(P, Q, D)


def _score_body(p_ref, h_ref, o_ref):
    pr = p_ref[...]
    inv_norm = jax.lax.rsqrt(jnp.sum(pr * pr))
    s = jax.lax.dot_general(pr, h_ref[...], (((1,), (1,)), ((), ())),
                            preferred_element_type=jnp.float32)
    o_ref[...] = jnp.tanh(s * inv_norm)


def _scores(h, p):
    n = h.shape[0]
    nb = n // _H
    return pl.pallas_call(
        _score_body,
        grid=(nb,),
        in_specs=[
            pl.BlockSpec((1, _H), lambda b: (0, 0)),
            pl.BlockSpec((_H, _H), lambda b: (b, 0)),
        ],
        out_specs=pl.BlockSpec((1, _H), lambda b: (0, b)),
        out_shape=jax.ShapeDtypeStruct((1, n), jnp.float32),
    )(p.reshape(1, _H), h)


def _topk_body(s_ref, ip_ref, *, n_real, k):
    nb = s_ref.shape[0]
    s = s_ref[...]
    rows = jax.lax.broadcasted_iota(jnp.int32, (nb, _H), 0)
    cols = jax.lax.broadcasted_iota(jnp.int32, (nb, _H), 1)
    gidx = rows * _H + cols
    valid = gidx < n_real
    u = jax.lax.bitcast_convert_type(s, jnp.int32)
    key = u ^ jnp.where(u < 0, 0x7FFFFFFF, 0)
    key = jnp.where(valid, key, _INT_MIN)

    def body(b, prefix):
        cand = jnp.where(b == 31, jnp.int32(0), prefix + (jnp.int32(1) << b))
        cnt = jnp.sum(jnp.where(key >= cand, 1.0, 0.0))
        return jnp.where(cnt >= k, cand, prefix)

    thr = jax.lax.fori_loop(0, 32, lambda i, c: body(31 - i, c),
                            jnp.int32(_INT_MIN), unroll=True)

    gt = jnp.where(key > thr, 1.0, 0.0)
    eq = jnp.where(key == thr, 1.0, 0.0)
    cnt_gt = jnp.sum(gt)
    r_need = k - cnt_gt

    upper = jnp.where(
        jax.lax.broadcasted_iota(jnp.int32, (_H, _H), 0)
        < jax.lax.broadcasted_iota(jnp.int32, (_H, _H), 1), 1.0, 0.0)
    lower_nb = jnp.where(
        jax.lax.broadcasted_iota(jnp.int32, (nb, nb), 1)
        < jax.lax.broadcasted_iota(jnp.int32, (nb, nb), 0), 1.0, 0.0)
    ones_col = jnp.ones((_H, 1), jnp.float32)

    def ex_prefix(m):
        within = jnp.dot(m, upper, preferred_element_type=jnp.float32)
        rowsum = jnp.dot(m, ones_col, preferred_element_type=jnp.float32)
        rowoff = jnp.dot(lower_nb, rowsum, preferred_element_type=jnp.float32)
        return within + rowoff

    eq_rank = ex_prefix(eq)
    sel = (gt > 0) | ((eq > 0) & (eq_rank < r_need))
    self32 = jnp.where(sel, 1.0, 0.0)
    slot = ex_prefix(self32)
    ip_ref[...] = jnp.where(sel, slot.astype(jnp.int32), -1)


def _topk_ip(s2d, n_real, k):
    nb = s2d.shape[0]
    return pl.pallas_call(
        functools.partial(_topk_body, n_real=n_real, k=k),
        in_specs=[pl.BlockSpec((nb, _H), lambda: (0, 0))],
        out_specs=pl.BlockSpec((nb, _H), lambda: (0, 0)),
        out_shape=jax.ShapeDtypeStruct((nb, _H), jnp.int32),
    )(s2d)


def _readout_body(u_ref, w_ref, b_ref, o_ref, acc, *, n_real, nb, bm):
    i = pl.program_id(0)

    @pl.when(i == 0)
    def _():
        acc[...] = jnp.zeros_like(acc)

    rows = i * bm + jax.lax.broadcasted_iota(jnp.int32, (bm, _H), 0)
    blk = jnp.where(rows < n_real, u_ref[...], 0.0)
    acc[...] += jnp.sum(blk, axis=0, keepdims=True)

    @pl.when(i == nb - 1)
    def _():
        g = acc[...] * (1.0 / n_real)
        o_ref[...] = jnp.dot(g, w_ref[...],
                             preferred_element_type=jnp.float32) + b_ref[...]


def _readout(u, Wo, bo, n_real, bm=512):
    n = u.shape[0]
    nb = n // bm
    return pl.pallas_call(
        functools.partial(_readout_body, n_real=n_real, nb=nb, bm=bm),
        grid=(nb,),
        in_specs=[
            pl.BlockSpec((bm, _H), lambda i: (i, 0)),
            pl.BlockSpec((_H, _H), lambda i: (0, 0)),
            pl.BlockSpec((1, _H), lambda i: (0, 0)),
        ],
        out_specs=pl.BlockSpec((1, _H), lambda i: (0, 0)),
        out_shape=jax.ShapeDtypeStruct((1, _H), jnp.float32),
        scratch_shapes=[pltpu.VMEM((1, _H), jnp.float32)],
    )(u, Wo, bo.reshape(1, _H))


def _sc_wid():
    return lax.axis_index("c") * 16 + lax.axis_index("s")


def _fill_1d(ref, n, val, dtype):
    for i in range(n // 16):
        ref[pl.ds(i * 16, 16)] = jnp.full((16,), val, dtype)


def _sc_deg(dst):
    ept = _E // _NW
    nwin = ept // 80
    rows = _NP // 16

    @functools.partial(
        pl.kernel,
        out_type=jax.ShapeDtypeStruct((2, _NP), jnp.float32),
        scratch_types=[
            pltpu.VMEM((80,), jnp.int32),
            pltpu.VMEM((80,), jnp.float32),
            pltpu.VMEM((rows,), jnp.float32),
            pltpu.VMEM_SHARED((_NP,), jnp.float32),
        ],
        **_SC_MESH,
    )
    def k(dst_hbm, out_hbm, idxw, ones_v, zbuf, degsh):
        cid = lax.axis_index("c")
        sid = lax.axis_index("s")
        _fill_1d(ones_v, 80, 1.0, jnp.float32)
        _fill_1d(zbuf, rows, 0.0, jnp.float32)
        pltpu.sync_copy(zbuf, degsh.at[pl.ds(sid * rows, rows)])
        plsc.subcore_barrier()
        base = (cid * 16 + sid) * ept

        def win(w, c):
            pltpu.sync_copy(dst_hbm.at[pl.ds(base + w * 80, 80)], idxw)
            pltpu.sync_copy(ones_v, degsh.at[idxw], add=True)
            return c

        lax.fori_loop(0, nwin, win, 0)
        plsc.subcore_barrier()
        pltpu.sync_copy(degsh.at[pl.ds(sid * rows, rows)],
                        out_hbm.at[cid, pl.ds(sid * rows, rows)])

    return k(dst)


def _sc_agg(y, src, dst):
    ept = _E // _NW
    nwin = ept // 80
    rows = _NP // 16

    @functools.partial(
        pl.kernel,
        out_type=jax.ShapeDtypeStruct((2, _NP, _H), jnp.float32),
        scratch_types=[
            pltpu.VMEM((80,), jnp.int32),
            pltpu.VMEM((80,), jnp.int32),
            pltpu.VMEM((80,), jnp.int32),
            pltpu.VMEM((80,), jnp.int32),
            pltpu.VMEM((80, _H), jnp.float32),
            pltpu.VMEM((80, _H), jnp.float32),
            pltpu.VMEM((16, _H), jnp.float32),
            pltpu.VMEM_SHARED((_NP, _H), jnp.float32),
            pltpu.SemaphoreType.DMA,
            pltpu.SemaphoreType.DMA,
            pltpu.SemaphoreType.DMA,
            pltpu.SemaphoreType.DMA,
        ],
        **_SC_MESH,
    )
    def k(y_hbm, src_hbm, dst_hbm, out_hbm, sidx, didx, sidx1, didx1,
          rbuf, rbuf1, zbuf, accsh, sem0, sem1, sem2, sem3):
        cid = lax.axis_index("c")
        sid = lax.axis_index("s")
        for r in range(16):
            _fill_1d(zbuf.at[r], _H, 0.0, jnp.float32)

        def zro(r, c):
            pltpu.sync_copy(zbuf, accsh.at[pl.ds(sid * rows + r * 16, 16)])
            return c

        lax.fori_loop(0, rows // 16, zro, 0)
        plsc.subcore_barrier()
        base = (cid * 16 + sid) * ept

        def win(w, c):
            pltpu.sync_copy(src_hbm.at[pl.ds(base + 2 * w * 80, 80)], sidx)
            pltpu.sync_copy(dst_hbm.at[pl.ds(base + 2 * w * 80, 80)], didx)
            g0 = pltpu.async_copy(y_hbm.at[sidx], rbuf, sem0)
            pltpu.sync_copy(src_hbm.at[pl.ds(base + (2 * w + 1) * 80, 80)],
                            sidx1)
            pltpu.sync_copy(dst_hbm.at[pl.ds(base + (2 * w + 1) * 80, 80)],
                            didx1)
            g1 = pltpu.async_copy(y_hbm.at[sidx1], rbuf1, sem1)
            g0.wait()
            s0 = pltpu.async_copy(rbuf, accsh.at[didx], sem2, add=True)
            g1.wait()
            s1 = pltpu.async_copy(rbuf1, accsh.at[didx1], sem3, add=True)
            s0.wait()
            s1.wait()
            return c

        lax.fori_loop(0, nwin // 2, win, 0)
        pltpu.sync_copy(src_hbm.at[pl.ds(base + (nwin - 1) * 80, 80)], sidx)
        pltpu.sync_copy(dst_hbm.at[pl.ds(base + (nwin - 1) * 80, 80)], didx)
        pltpu.async_copy(y_hbm.at[sidx], rbuf, sem0).wait()
        pltpu.sync_copy(rbuf, accsh.at[didx], add=True)
        plsc.subcore_barrier()
        pltpu.sync_copy(accsh.at[pl.ds(sid * rows, rows)],
                        out_hbm.at[cid, pl.ds(sid * rows, rows)])

    return k(y, src, dst)


def _sc_compact(ip_flat, s_flat, n, kp, zero_row):
    nwin = n // 2048

    @functools.partial(
        pl.kernel,
        out_type=(jax.ShapeDtypeStruct((kp,), jnp.int32),
                  jax.ShapeDtypeStruct((kp,), jnp.float32)),
        scratch_types=[
            pltpu.VMEM((2048,), jnp.int32),
            pltpu.VMEM((2048,), jnp.float32),
            pltpu.VMEM((kp,), jnp.int32),
            pltpu.VMEM((kp,), jnp.float32),
        ],
        **_SC_MESH,
    )
    def k(ip_hbm, s_hbm, perm_hbm, vals_hbm, ipw, sw, permb, valsb):
        wid = _sc_wid()

        @pl.when(wid == 0)
        def _():
            _fill_1d(permb, kp, zero_row, jnp.int32)
            _fill_1d(valsb, kp, 0.0, jnp.float32)

            def win(w, c):
                pltpu.sync_copy(ip_hbm.at[pl.ds(w * 2048, 2048)], ipw)
                pltpu.sync_copy(s_hbm.at[pl.ds(w * 2048, 2048)], sw)

                def inner(i, c2):
                    idx = ipw[pl.ds(i * 16, 16)]
                    msk = idx >= 0
                    idx2 = jnp.maximum(idx, 0)
                    gi = (w * 2048 + i * 16
                          + lax.iota(jnp.int32, 16))
                    plsc.store_scatter(permb, [idx2], gi, mask=msk)
                    plsc.store_scatter(valsb, [idx2], sw[pl.ds(i * 16, 16)],
                                       mask=msk)
                    return c2

                lax.fori_loop(0, 128, inner, 0)
                return c

            lax.fori_loop(0, nwin, win, 0)
            pltpu.sync_copy(permb, perm_hbm)
            pltpu.sync_copy(valsb, vals_hbm)

    return k(ip_flat, s_flat)


def _sc_rowgather(T, idx, clamp=False):
    kp = idx.shape[0]
    w = T.shape[1]
    rpw = kp // _NW
    nwin = rpw // 16

    @functools.partial(
        pl.kernel,
        out_type=jax.ShapeDtypeStruct((kp, w), T.dtype),
        scratch_types=[
            pltpu.VMEM((16,), jnp.int32),
            pltpu.VMEM((16, w), T.dtype),
            pltpu.SemaphoreType.DMA,
        ],
        **_SC_MESH,
    )
    def k(t_hbm, idx_hbm, out_hbm, idxw, rbuf, sem):
        base = _sc_wid() * rpw

        def win(r, c):
            pltpu.sync_copy(idx_hbm.at[pl.ds(base + r * 16, 16)], idxw)
            if clamp:
                idxw[pl.ds(0, 16)] = jnp.maximum(idxw[pl.ds(0, 16)], 0)
            pltpu.async_copy(t_hbm.at[idxw], rbuf, sem).wait()
            pltpu.sync_copy(rbuf, out_hbm.at[pl.ds(base + r * 16, 16)])
            return c

        lax.fori_loop(0, nwin, win, 0)

    return k(T, idx)


def _sc_colgather(T, idx):
    m, w = T.shape
    kp = idx.shape[0]
    rpw = m // _NW

    @functools.partial(
        pl.kernel,
        out_type=jax.ShapeDtypeStruct((m, kp), jnp.float32),
        scratch_types=[
            pltpu.VMEM((kp,), jnp.int32),
            pltpu.VMEM((w,), jnp.float32),
            pltpu.VMEM((kp,), jnp.float32),
        ],
        **_SC_MESH,
    )
    def k(t_hbm, idx_hbm, out_hbm, idxall, rowb, outb):
        base = _sc_wid() * rpw
        pltpu.sync_copy(idx_hbm, idxall)

        def row(r, c):
            pltpu.sync_copy(t_hbm.at[base + r], rowb)

            def inner(j, c2):
                iv = idxall[pl.ds(j * 16, 16)]
                outb[pl.ds(j * 16, 16)] = plsc.load_gather(rowb, [iv])
                return c2

            lax.fori_loop(0, kp // 16, inner, 0)
            pltpu.sync_copy(outb, out_hbm.at[base + r])
            return c

        lax.fori_loop(0, rpw, row, 0)

    return k(T, idx)


_CHUNK = 819200
_EPT3 = _E // _NW
_EPT3P = 10240


def _sc_build_bcd(src, dst, ip):
    nwin = _EPT3P // 128

    @functools.partial(
        pl.kernel,
        out_type=(jax.ShapeDtypeStruct((_NP * _K1P,), jnp.float32),
                  jax.ShapeDtypeStruct((_K1P * _NP,), jnp.float32),
                  jax.ShapeDtypeStruct((_K1P * _K1P,), jnp.float32)),
        scratch_types=[
            pltpu.VMEM((_EPT3P,), jnp.int32),
            pltpu.VMEM((_EPT3P,), jnp.int32),
            pltpu.VMEM((_EPT3P,), jnp.int32),
            pltpu.VMEM((_EPT3P,), jnp.int32),
            pltpu.VMEM((_NP,), jnp.int32),
            pltpu.VMEM((10240,), jnp.float32),
            pltpu.VMEM((128,), jnp.int32),
            pltpu.VMEM((128,), jnp.int32),
            pltpu.VMEM((128,), jnp.float32),
            pltpu.VMEM_SHARED((_CHUNK + 16,), jnp.float32),
            pltpu.SemaphoreType.DMA,
            pltpu.SemaphoreType.DMA,
        ],
        **_SC_MESH,
    )
    def k(src_hbm, dst_hbm, ip_hbm, b_hbm, c_hbm, d_hbm,
          srcc, dstc, ipsc, ipdc, iptab, zbuf, idxw, idxw1, ones_v, chunk,
          sem0, sem1):
        cid = lax.axis_index("c")
        sid = lax.axis_index("s")
        ebase = (cid * 16 + sid) * _EPT3
        pltpu.sync_copy(src_hbm.at[pl.ds(ebase, _EPT3)],
                        srcc.at[pl.ds(0, _EPT3)])
        pltpu.sync_copy(dst_hbm.at[pl.ds(ebase, _EPT3)],
                        dstc.at[pl.ds(0, _EPT3)])
        pltpu.sync_copy(ip_hbm, iptab)

        def fill(i, c):
            zbuf[pl.ds(i * 16, 16)] = jnp.zeros((16,), jnp.float32)
            return c

        lax.fori_loop(0, 10240 // 16, fill, 0)
        _fill_1d(ones_v, 128, 1.0, jnp.float32)

        def pre(i, c):
            gi = i * 16 + lax.iota(jnp.int32, 16)
            inb = gi < _EPT3
            s = jnp.where(inb, srcc[pl.ds(i * 16, 16)], 0)
            d = jnp.where(inb, dstc[pl.ds(i * 16, 16)], 0)
            ok = inb & (s != d)
            ipsc[pl.ds(i * 16, 16)] = jnp.where(
                ok, plsc.load_gather(iptab, [s]), -1)
            ipdc[pl.ds(i * 16, 16)] = jnp.where(
                ok, plsc.load_gather(iptab, [d]), -1)
            return c

        lax.fori_loop(0, _EPT3P // 16, pre, 0)

        tel = sid * 51200

        def phase(rowarr, colarr, out_hbm, rows, width, nchunks):
            def one_chunk(j, c):
                r0 = (cid * nchunks + j) * rows

                def zro(z, c2):
                    pltpu.sync_copy(
                        zbuf, chunk.at[pl.ds(tel + z * 10240, 10240)])
                    return c2

                lax.fori_loop(0, 5, zro, 0)
                plsc.subcore_barrier()

                dump = _CHUNK + lax.iota(jnp.int32, 16)

                def build(w, buf):
                    for v in range(8):
                        sl = pl.ds(w * 128 + v * 16, 16)
                        rv = rowarr[sl]
                        cv = colarr[sl]
                        ok = (rv >= r0) & (rv < r0 + rows) & (cv >= 0)
                        lidx = (rv - r0) * width + cv
                        buf[pl.ds(v * 16, 16)] = jnp.where(ok, lidx, dump)

                def win(w, c2):
                    build(2 * w, idxw)
                    cp0 = pltpu.async_copy(ones_v, chunk.at[idxw], sem0,
                                           add=True)
                    build(2 * w + 1, idxw1)
                    cp1 = pltpu.async_copy(ones_v, chunk.at[idxw1], sem1,
                                           add=True)
                    cp0.wait()
                    cp1.wait()
                    return c2

                lax.fori_loop(0, nwin // 2, win, 0)
                plsc.subcore_barrier()
                pltpu.sync_copy(
                    chunk.at[pl.ds(tel, 51200)],
                    out_hbm.at[pl.ds(r0 * width + tel, 51200)])
                plsc.subcore_barrier()
                return c

            lax.fori_loop(0, nchunks, one_chunk, 0)

        phase(srcc, ipdc, b_hbm, 160, _K1P, 32)
        phase(ipsc, dstc, c_hbm, 80, _NP, 32)
        phase(ipsc, ipdc, d_hbm, 160, _K1P, 16)

    return k(src, dst, ip)


def _ew_gcn0_body(p0_ref, p1_ref, y_ref, d_ref, b_ref, o_ref, *, act):
    r = (p0_ref[...] + p1_ref[...] + 2.0 * y_ref[...]) * d_ref[...] + b_ref[...]
    if act:
        r = jnp.where(r > 0, r, jnp.exp(r) - 1.0)
    o_ref[...] = r


def _ew_gcn0(P0, P1, y, dinv_col, b, act, bm=512):
    n = y.shape[0]
    return pl.pallas_call(
        functools.partial(_ew_gcn0_body, act=act),
        grid=(n // bm,),
        in_specs=[pl.BlockSpec((bm, _H), lambda i: (i, 0))] * 3
        + [pl.BlockSpec((bm, 1), lambda i: (i, 0)),
           pl.BlockSpec((1, _H), lambda i: (0, 0))],
        out_specs=pl.BlockSpec((bm, _H), lambda i: (i, 0)),
        out_shape=jax.ShapeDtypeStruct((n, _H), jnp.float32),
    )(P0, P1, y, dinv_col, b.reshape(1, _H))


def _mask_add_body(res_ref, g_ref, ip_ref, o_ref):
    o_ref[...] = res_ref[...] + jnp.where(ip_ref[...] >= 0, g_ref[...], 0.0)


def _mask_add(res, g, ip_col, bm=None):
    n = res.shape[0]
    bm = bm or _bs(n)
    return pl.pallas_call(
        _mask_add_body,
        grid=(n // bm,),
        in_specs=[pl.BlockSpec((bm, _H), lambda i: (i, 0)),
                  pl.BlockSpec((bm, _H), lambda i: (i, 0)),
                  pl.BlockSpec((bm, 1), lambda i: (i, 0))],
        out_specs=pl.BlockSpec((bm, _H), lambda i: (i, 0)),
        out_shape=jax.ShapeDtypeStruct((n, _H), jnp.float32),
    )(res, g, ip_col)


def _adj_comb_body(a_ref, d_ref, o_ref, *, bm, bn):
    i = pl.program_id(0)
    j = pl.program_id(1)
    r = a_ref[...] + 2.0 * d_ref[...]
    rows = i * bm + jax.lax.broadcasted_iota(jnp.int32, (bm, bn), 0)
    cols = j * bn + jax.lax.broadcasted_iota(jnp.int32, (bm, bn), 1)
    o_ref[...] = jnp.where(rows == cols, 0.0, r)


def _adj_combine(A, D, bm=None, bn=None):
    m, n = A.shape
    bm = bm or _bs(m)
    bn = bn or _bs(n)
    return pl.pallas_call(
        functools.partial(_adj_comb_body, bm=bm, bn=bn),
        grid=(m // bm, n // bn),
        in_specs=[pl.BlockSpec((bm, bn), lambda i, j: (i, j)),
                  pl.BlockSpec((bm, bn), lambda i, j: (i, j))],
        out_specs=pl.BlockSpec((bm, bn), lambda i, j: (i, j)),
        out_shape=jax.ShapeDtypeStruct((m, n), jnp.float32),
    )(A, D)


def _topk_stage(h, p, n_real, k, kp):
    s = _scores(h, p)
    np_ = h.shape[0]
    ip2d = _topk_ip(s.reshape(-1, _H), n_real, k)
    ip = ip2d.reshape(-1)
    perm, vals = _sc_compact(ip, s.reshape(-1), np_, kp, zero_row=n_real)
    return perm, vals, ip


def _restricted_square(M, perm):
    C = _sc_rowgather(M, perm)
    D = _sc_colgather(C, perm)
    G = _big_mm(C, M)
    Gc = _sc_colgather(G, perm)
    return _adj_combine(Gc, D)


def _gcn_dense(M, x, W, b, act, extra=None):
    deg = _colsum(M) + 2.0
    dinv_col = jax.lax.rsqrt(deg).reshape(-1, 1)
    scale = dinv_col if extra is None else dinv_col * extra
    y = _feat_mm(x, W, jnp.zeros((_H,), jnp.float32), scale=scale)
    return _gcn_agg(M, y, dinv_col, b, act)


def kernel(x, edge_index, batch, Wd0, bd0, Wd1, bd1, Wd2, bd2, Wd3, bd3,
           p1, p2, p3, Wu0, bu0, Wu1, bu1, Wu2, bu2, Wo, bo):
    src = edge_index[0]
    dst = edge_index[1]
    xp = jnp.pad(x, ((0, _NP - _N), (0, 0)))

    degp = _sc_deg(dst)
    deg0 = degp[0] + degp[1] + 2.0
    dinv0c = jax.lax.rsqrt(deg0).reshape(-1, 1)

    def gcn_edges(xin, W, b, act):
        y = _feat_mm(xin, W, jnp.zeros((_H,), jnp.float32), scale=dinv0c)
        P = _sc_agg(y, src, dst)
        return _ew_gcn0(P[0], P[1], y, dinv0c, b, act)

    h0 = gcn_edges(xp, Wd0, bd0, True)

    perm1, vals1, ip1 = _topk_stage(h0, p1, _N, _K1, _K1P)
    Bf, Cf, Df = _sc_build_bcd(src, dst, ip1)
    M1 = _big_mm(Cf.reshape(_K1P, _NP), Bf.reshape(_NP, _K1P),
                 D=Df.reshape(_K1P, _K1P), diag_zero=True)
    h1 = _gcn_dense(M1, _sc_rowgather(h0, perm1), Wd1, bd1, True,
                    extra=vals1.reshape(-1, 1))

    perm2, vals2, ip2 = _topk_stage(h1, p2, _K1, _K2, _K2P)
    M2 = _restricted_square(M1, perm2)
    h2 = _gcn_dense(M2, _sc_rowgather(h1, perm2), Wd2, bd2, True,
                    extra=vals2.reshape(-1, 1))

    perm3, vals3, ip3 = _topk_stage(h2, p3, _K2, _K3, _K3P)
    M3 = _restricted_square(M2, perm3)
    h3 = _gcn_dense(M3, _sc_rowgather(h2, perm3), Wd3, bd3, True,
                    extra=vals3.reshape(-1, 1))

    u = _mask_add(h2, _sc_rowgather(h3, ip3, clamp=True), ip3.reshape(-1, 1))
    u = _gcn_dense(M2, u, Wu0, bu0, True)
    u = _mask_add(h1, _sc_rowgather(u, ip2, clamp=True), ip2.reshape(-1, 1))
    u = _gcn_dense(M1, u, Wu1, bu1, True)
    u = _mask_add(h0, _sc_rowgather(u, ip1, clamp=True), ip1.reshape(-1, 1))
    u = gcn_edges(u, Wu2, bu2, False)

    return _readout(u, Wo, bo, _N)


# speedup vs baseline: 2.1868x; 1.2761x over previous
import functools
import math

import jax
import jax.numpy as jnp
from jax import lax
from jax.experimental import pallas as pl
from jax.experimental.pallas import tpu as pltpu
from jax.experimental.pallas import tpu_sc as plsc

_SC_MESH = dict(mesh=plsc.VectorSubcoreMesh(core_axis_name="c",
                                            subcore_axis_name="s"),
                compiler_params=pltpu.CompilerParams(
                    needs_layout_passes=False))
_NW = 32

_N = 10000
_E = 320000
_H = 128
_NP = 10240
_K1, _K1P = 5000, 5120
_K2, _K2P = 2500, 2560
_K3, _K3P = 1250, 1280

_INT_MIN = -2147483648


def _bs(n):
    return 512 if n % 512 == 0 else 256


def _bsl(n):
    return 1024 if n % 1024 == 0 else _bs(n)


def _feat_mm_body(x_ref, w_ref, s_ref, b_ref, o_ref, *, act, use_scale):
    acc = jnp.dot(x_ref[...], w_ref[...], preferred_element_type=jnp.float32)
    if use_scale:
        acc = acc * s_ref[...]
    acc = acc + b_ref[...]
    if act:
        acc = jnp.where(acc > 0, acc, jnp.exp(acc) - 1.0)
    o_ref[...] = acc


def _feat_mm(x, W, b, scale=None, act=False, bm=None):
    n = x.shape[0]
    bm = bm or _bs(n)
    use_scale = scale is not None
    if scale is None:
        scale = jnp.zeros((n, 1), jnp.float32)
    grid = (n // bm,)
    return pl.pallas_call(
        functools.partial(_feat_mm_body, act=act, use_scale=use_scale),
        grid=grid,
        in_specs=[
            pl.BlockSpec((bm, _H), lambda i: (i, 0)),
            pl.BlockSpec((_H, _H), lambda i: (0, 0)),
            pl.BlockSpec((bm, 1), lambda i: (i, 0)),
            pl.BlockSpec((1, _H), lambda i: (0, 0)),
        ],
        out_specs=pl.BlockSpec((bm, _H), lambda i: (i, 0)),
        out_shape=jax.ShapeDtypeStruct((n, _H), jnp.float32),
    )(x, W, scale, b.reshape(1, _H))


def _colsum_body(m_ref, o_ref):
    i = pl.program_id(1)

    @pl.when(i == 0)
    def _():
        o_ref[...] = jnp.zeros_like(o_ref)

    o_ref[...] += jnp.sum(m_ref[...], axis=0, keepdims=True)


def _colsum(M, bi=None, bj=None):
    n, m = M.shape
    bi = bi or _bs(n)
    bj = bj or _bs(m)
    return pl.pallas_call(
        _colsum_body,
        grid=(m // bj, n // bi),
        in_specs=[pl.BlockSpec((bi, bj), lambda j, i: (i, j))],
        out_specs=pl.BlockSpec((1, bj), lambda j, i: (0, j)),
        out_shape=jax.ShapeDtypeStruct((1, m), jnp.float32),
    )(M)


def _agg_body(m_ref, y_ref, y2_ref, d_ref, b_ref, o_ref, acc, *, nk, act):
    k = pl.program_id(1)

    @pl.when(k == 0)
    def _():
        acc[...] = jnp.zeros_like(acc)

    acc[...] += jax.lax.dot_general(
        m_ref[...], y_ref[...], (((0,), (0,)), ((), ())),
        preferred_element_type=jnp.float32)

    @pl.when(k == nk - 1)
    def _():
        r = (acc[...] + 2.0 * y2_ref[...]) * d_ref[...] + b_ref[...]
        if act:
            r = jnp.where(r > 0, r, jnp.exp(r) - 1.0)
        o_ref[...] = r


def _gcn_agg(M, y, dinv_col, b, act, bi=None, bk=None):
    n = M.shape[0]
    bi = bi or _bs(n)
    bk = bk or _bs(n)
    return pl.pallas_call(
        functools.partial(_agg_body, nk=n // bk, act=act),
        grid=(n // bi, n // bk),
        in_specs=[
            pl.BlockSpec((bk, bi), lambda i, k: (k, i)),
            pl.BlockSpec((bk, _H), lambda i, k: (k, 0)),
            pl.BlockSpec((bi, _H), lambda i, k: (i, 0)),
            pl.BlockSpec((bi, 1), lambda i, k: (i, 0)),
            pl.BlockSpec((1, _H), lambda i, k: (0, 0)),
        ],
        out_specs=pl.BlockSpec((bi, _H), lambda i, k: (i, 0)),
        out_shape=jax.ShapeDtypeStruct((n, _H), jnp.float32),
        scratch_shapes=[pltpu.VMEM((bi, _H), jnp.float32)],
    )(M, y, y, dinv_col, b.reshape(1, _H))


def _gcn_dense(M, x, W, b, act):
    deg = _colsum(M) + 2.0
    dinv_col = jax.lax.rsqrt(deg).reshape(-1, 1)
    y = _feat_mm(x, W, jnp.zeros((_H,), jnp.float32), scale=dinv_col)
    return _gcn_agg(M, y, dinv_col, b, act)


def _bigmm_body(p_ref, q_ref, d_ref, o_ref, acc, *, nk, use_d, diag_zero, bm, bn):
    mi = pl.program_id(0)
    nj = pl.program_id(1)
    k = pl.program_id(2)

    @pl.when(k == 0)
    def _():
        acc[...] = jnp.zeros_like(acc)

    acc[...] += jnp.dot(p_ref[...].astype(jnp.bfloat16),
                        q_ref[...].astype(jnp.bfloat16),
                        preferred_element_type=jnp.float32)

    @pl.when(k == nk - 1)
    def _():
        r = acc[...]
        if use_d:
            r = r + 2.0 * d_ref[...]
        if diag_zero:
            rows = mi * bm + jax.lax.broadcasted_iota(jnp.int32, (bm, bn), 0)
            cols = nj * bn + jax.lax.broadcasted_iota(jnp.int32, (bm, bn), 1)
            r = jnp.where(rows == cols, 0.0, r)
        o_ref[...] = r


def _big_mm(P, Q, D=None, diag_zero=False, bm=None, bn=None, bk=None):
    m, kdim = P.shape
    n = Q.shape[1]
    bm = bm or _bsl(m)
    bn = bn or _bsl(n)
    bk = bk or _bsl(kdim)
    use_d = D is not None
    if D is None:
        D = jnp.zeros((bm, bn), jnp.float32)
        d_spec = pl.BlockSpec((bm, bn), lambda i, j, k: (0, 0))
    else:
        d_spec = pl.BlockSpec((bm, bn), lambda i, j, k: (i, j))
    return pl.pallas_call(
        functools.partial(_bigmm_body, nk=kdim // bk, use_d=use_d,
                          diag_zero=diag_zero, bm=bm, bn=bn),
        grid=(m // bm, n // bn, kdim // bk),
        in_specs=[
            pl.BlockSpec((bm, bk), lambda i, j, k: (i, k)),
            pl.BlockSpec((bk, bn), lambda i, j, k: (k, j)),
            d_spec,
        ],
        out_specs=pl.BlockSpec((bm, bn), lambda i, j, k: (i, j)),
        out_shape=jax.ShapeDtypeStruct((m, n), jnp.float32),
        scratch_shapes=[pltpu.VMEM((bm, bn), jnp.float32)],
    )(P, Q, D)


def _score_body(p_ref, h_ref, o_ref):
    pr = p_ref[...]
    inv_norm = jax.lax.rsqrt(jnp.sum(pr * pr))
    s = jax.lax.dot_general(pr, h_ref[...], (((1,), (1,)), ((), ())),
                            preferred_element_type=jnp.float32)
    o_ref[...] = jnp.tanh(s * inv_norm)


def _scores(h, p):
    n = h.shape[0]
    nb = n // _H
    return pl.pallas_call(
        _score_body,
        grid=(nb,),
        in_specs=[
            pl.BlockSpec((1, _H), lambda b: (0, 0)),
            pl.BlockSpec((_H, _H), lambda b: (b, 0)),
        ],
        out_specs=pl.BlockSpec((1, _H), lambda b: (0, b)),
        out_shape=jax.ShapeDtypeStruct((1, n), jnp.float32),
    )(p.reshape(1, _H), h)


def _topk_body(s_ref, ip_ref, *, n_real, k):
    nb = s_ref.shape[0]
    s = s_ref[...]
    rows = jax.lax.broadcasted_iota(jnp.int32, (nb, _H), 0)
    cols = jax.lax.broadcasted_iota(jnp.int32, (nb, _H), 1)
    gidx = rows * _H + cols
    valid = gidx < n_real
    u = jax.lax.bitcast_convert_type(s, jnp.int32)
    key = u ^ jnp.where(u < 0, 0x7FFFFFFF, 0)
    key = jnp.where(valid, key, _INT_MIN)

    def body(b, prefix):
        cand = jnp.where(b == 31, jnp.int32(0), prefix + (jnp.int32(1) << b))
        cnt = jnp.sum(jnp.where(key >= cand, 1.0, 0.0))
        return jnp.where(cnt >= k, cand, prefix)

    thr = jax.lax.fori_loop(0, 32, lambda i, c: body(31 - i, c),
                            jnp.int32(_INT_MIN), unroll=True)

    gt = jnp.where(key > thr, 1.0, 0.0)
    eq = jnp.where(key == thr, 1.0, 0.0)
    cnt_gt = jnp.sum(gt)
    r_need = k - cnt_gt

    upper = jnp.where(
        jax.lax.broadcasted_iota(jnp.int32, (_H, _H), 0)
        < jax.lax.broadcasted_iota(jnp.int32, (_H, _H), 1), 1.0, 0.0)
    lower_nb = jnp.where(
        jax.lax.broadcasted_iota(jnp.int32, (nb, nb), 1)
        < jax.lax.broadcasted_iota(jnp.int32, (nb, nb), 0), 1.0, 0.0)
    ones_col = jnp.ones((_H, 1), jnp.float32)

    def ex_prefix(m):
        within = jnp.dot(m, upper, preferred_element_type=jnp.float32)
        rowsum = jnp.dot(m, ones_col, preferred_element_type=jnp.float32)
        rowoff = jnp.dot(lower_nb, rowsum, preferred_element_type=jnp.float32)
        return within + rowoff

    eq_rank = ex_prefix(eq)
    sel = (gt > 0) | ((eq > 0) & (eq_rank < r_need))
    self32 = jnp.where(sel, 1.0, 0.0)
    slot = ex_prefix(self32)
    ip_ref[...] = jnp.where(sel, slot.astype(jnp.int32), -1)


def _topk_ip(s2d, n_real, k):
    nb = s2d.shape[0]
    return pl.pallas_call(
        functools.partial(_topk_body, n_real=n_real, k=k),
        in_specs=[pl.BlockSpec((nb, _H), lambda: (0, 0))],
        out_specs=pl.BlockSpec((nb, _H), lambda: (0, 0)),
        out_shape=jax.ShapeDtypeStruct((nb, _H), jnp.int32),
    )(s2d)


def _readout_body(u_ref, w_ref, b_ref, o_ref, acc, *, n_real, nb, bm):
    i = pl.program_id(0)

    @pl.when(i == 0)
    def _():
        acc[...] = jnp.zeros_like(acc)

    rows = i * bm + jax.lax.broadcasted_iota(jnp.int32, (bm, _H), 0)
    blk = jnp.where(rows < n_real, u_ref[...], 0.0)
    acc[...] += jnp.sum(blk, axis=0, keepdims=True)

    @pl.when(i == nb - 1)
    def _():
        g = acc[...] * (1.0 / n_real)
        o_ref[...] = jnp.dot(g, w_ref[...],
                             preferred_element_type=jnp.float32) + b_ref[...]


def _readout(u, Wo, bo, n_real, bm=512):
    n = u.shape[0]
    nb = n // bm
    return pl.pallas_call(
        functools.partial(_readout_body, n_real=n_real, nb=nb, bm=bm),
        grid=(nb,),
        in_specs=[
            pl.BlockSpec((bm, _H), lambda i: (i, 0)),
            pl.BlockSpec((_H, _H), lambda i: (0, 0)),
            pl.BlockSpec((1, _H), lambda i: (0, 0)),
        ],
        out_specs=pl.BlockSpec((1, _H), lambda i: (0, 0)),
        out_shape=jax.ShapeDtypeStruct((1, _H), jnp.float32),
        scratch_shapes=[pltpu.VMEM((1, _H), jnp.float32)],
    )(u, Wo, bo.reshape(1, _H))


def _sc_wid():
    return lax.axis_index("c") * 16 + lax.axis_index("s")


def _fill_1d(ref, n, val, dtype):
    for i in range(n // 16):
        ref[pl.ds(i * 16, 16)] = jnp.full((16,), val, dtype)


def _sc_deg(dst):
    ept = _E // _NW
    nwin = ept // 80
    rows = _NP // 16

    @functools.partial(
        pl.kernel,
        out_type=jax.ShapeDtypeStruct((2, _NP), jnp.float32),
        scratch_types=[
            pltpu.VMEM((80,), jnp.int32),
            pltpu.VMEM((80,), jnp.float32),
            pltpu.VMEM((rows,), jnp.float32),
            pltpu.VMEM_SHARED((_NP,), jnp.float32),
        ],
        **_SC_MESH,
    )
    def k(dst_hbm, out_hbm, idxw, ones_v, zbuf, degsh):
        cid = lax.axis_index("c")
        sid = lax.axis_index("s")
        _fill_1d(ones_v, 80, 1.0, jnp.float32)
        _fill_1d(zbuf, rows, 0.0, jnp.float32)
        pltpu.sync_copy(zbuf, degsh.at[pl.ds(sid * rows, rows)])
        plsc.subcore_barrier()
        base = (cid * 16 + sid) * ept

        def win(w, c):
            pltpu.sync_copy(dst_hbm.at[pl.ds(base + w * 80, 80)], idxw)
            pltpu.sync_copy(ones_v, degsh.at[idxw], add=True)
            return c

        lax.fori_loop(0, nwin, win, 0)
        plsc.subcore_barrier()
        pltpu.sync_copy(degsh.at[pl.ds(sid * rows, rows)],
                        out_hbm.at[cid, pl.ds(sid * rows, rows)])

    return k(dst)


def _sc_agg(y, src, dst):
    ept = _E // _NW
    nwin = ept // 80
    rows = _NP // 16

    @functools.partial(
        pl.kernel,
        out_type=jax.ShapeDtypeStruct((2, _NP, _H), jnp.float32),
        scratch_types=[
            pltpu.VMEM((80,), jnp.int32),
            pltpu.VMEM((80,), jnp.int32),
            pltpu.VMEM((80,), jnp.int32),
            pltpu.VMEM((80,), jnp.int32),
            pltpu.VMEM((80, _H), jnp.float32),
            pltpu.VMEM((80, _H), jnp.float32),
            pltpu.VMEM((16, _H), jnp.float32),
            pltpu.VMEM_SHARED((_NP, _H), jnp.float32),
            pltpu.SemaphoreType.DMA,
            pltpu.SemaphoreType.DMA,
            pltpu.SemaphoreType.DMA,
            pltpu.SemaphoreType.DMA,
        ],
        **_SC_MESH,
    )
    def k(y_hbm, src_hbm, dst_hbm, out_hbm, sidx, didx, sidx1, didx1,
          rbuf, rbuf1, zbuf, accsh, sem0, sem1, sem2, sem3):
        cid = lax.axis_index("c")
        sid = lax.axis_index("s")
        for r in range(16):
            _fill_1d(zbuf.at[r], _H, 0.0, jnp.float32)

        def zro(r, c):
            pltpu.sync_copy(zbuf, accsh.at[pl.ds(sid * rows + r * 16, 16)])
            return c

        lax.fori_loop(0, rows // 16, zro, 0)
        plsc.subcore_barrier()
        base = (cid * 16 + sid) * ept

        def win(w, c):
            pltpu.sync_copy(src_hbm.at[pl.ds(base + 2 * w * 80, 80)], sidx)
            pltpu.sync_copy(dst_hbm.at[pl.ds(base + 2 * w * 80, 80)], didx)
            g0 = pltpu.async_copy(y_hbm.at[sidx], rbuf, sem0)
            pltpu.sync_copy(src_hbm.at[pl.ds(base + (2 * w + 1) * 80, 80)],
                            sidx1)
            pltpu.sync_copy(dst_hbm.at[pl.ds(base + (2 * w + 1) * 80, 80)],
                            didx1)
            g1 = pltpu.async_copy(y_hbm.at[sidx1], rbuf1, sem1)
            g0.wait()
            s0 = pltpu.async_copy(rbuf, accsh.at[didx], sem2, add=True)
            g1.wait()
            s1 = pltpu.async_copy(rbuf1, accsh.at[didx1], sem3, add=True)
            s0.wait()
            s1.wait()
            return c

        lax.fori_loop(0, nwin // 2, win, 0)
        pltpu.sync_copy(src_hbm.at[pl.ds(base + (nwin - 1) * 80, 80)], sidx)
        pltpu.sync_copy(dst_hbm.at[pl.ds(base + (nwin - 1) * 80, 80)], didx)
        pltpu.async_copy(y_hbm.at[sidx], rbuf, sem0).wait()
        pltpu.sync_copy(rbuf, accsh.at[didx], add=True)
        plsc.subcore_barrier()
        pltpu.sync_copy(accsh.at[pl.ds(sid * rows, rows)],
                        out_hbm.at[cid, pl.ds(sid * rows, rows)])

    return k(y, src, dst)


def _sc_compact(ip_flat, s_flat, n, kp, zero_row):
    nwin = n // 2048

    @functools.partial(
        pl.kernel,
        out_type=(jax.ShapeDtypeStruct((kp,), jnp.int32),
                  jax.ShapeDtypeStruct((kp,), jnp.float32)),
        scratch_types=[
            pltpu.VMEM((2048,), jnp.int32),
            pltpu.VMEM((2048,), jnp.float32),
            pltpu.VMEM((kp,), jnp.int32),
            pltpu.VMEM((kp,), jnp.float32),
        ],
        **_SC_MESH,
    )
    def k(ip_hbm, s_hbm, perm_hbm, vals_hbm, ipw, sw, permb, valsb):
        wid = _sc_wid()

        @pl.when(wid == 0)
        def _():
            _fill_1d(permb, kp, zero_row, jnp.int32)
            _fill_1d(valsb, kp, 0.0, jnp.float32)

            def win(w, c):
                pltpu.sync_copy(ip_hbm.at[pl.ds(w * 2048, 2048)], ipw)
                pltpu.sync_copy(s_hbm.at[pl.ds(w * 2048, 2048)], sw)

                def inner(i, c2):
                    idx = ipw[pl.ds(i * 16, 16)]
                    msk = idx >= 0
                    idx2 = jnp.maximum(idx, 0)
                    gi = (w * 2048 + i * 16
                          + lax.iota(jnp.int32, 16))
                    plsc.store_scatter(permb, [idx2], gi, mask=msk)
                    plsc.store_scatter(valsb, [idx2], sw[pl.ds(i * 16, 16)],
                                       mask=msk)
                    return c2

                lax.fori_loop(0, 128, inner, 0)
                return c

            lax.fori_loop(0, nwin, win, 0)
            pltpu.sync_copy(permb, perm_hbm)
            pltpu.sync_copy(valsb, vals_hbm)

    return k(ip_flat, s_flat)


def _sc_rowgather(T, idx, clamp=False):
    kp = idx.shape[0]
    w = T.shape[1]
    rpw = kp // _NW
    nwin = rpw // 16

    @functools.partial(
        pl.kernel,
        out_type=jax.ShapeDtypeStruct((kp, w), T.dtype),
        scratch_types=[
            pltpu.VMEM((16,), jnp.int32),
            pltpu.VMEM((16, w), T.dtype),
            pltpu.SemaphoreType.DMA,
        ],
        **_SC_MESH,
    )
    def k(t_hbm, idx_hbm, out_hbm, idxw, rbuf, sem):
        base = _sc_wid() * rpw

        def win(r, c):
            pltpu.sync_copy(idx_hbm.at[pl.ds(base + r * 16, 16)], idxw)
            if clamp:
                idxw[pl.ds(0, 16)] = jnp.maximum(idxw[pl.ds(0, 16)], 0)
            pltpu.async_copy(t_hbm.at[idxw], rbuf, sem).wait()
            pltpu.sync_copy(rbuf, out_hbm.at[pl.ds(base + r * 16, 16)])
            return c

        lax.fori_loop(0, nwin, win, 0)

    return k(T, idx)


def _sc_colgather(T, idx):
    m, w = T.shape
    kp = idx.shape[0]
    rpw = m // _NW

    @functools.partial(
        pl.kernel,
        out_type=jax.ShapeDtypeStruct((m, kp), jnp.float32),
        scratch_types=[
            pltpu.VMEM((kp,), jnp.int32),
            pltpu.VMEM((w,), jnp.float32),
            pltpu.VMEM((kp,), jnp.float32),
        ],
        **_SC_MESH,
    )
    def k(t_hbm, idx_hbm, out_hbm, idxall, rowb, outb):
        base = _sc_wid() * rpw
        pltpu.sync_copy(idx_hbm, idxall)

        def row(r, c):
            pltpu.sync_copy(t_hbm.at[base + r], rowb)

            def inner(j, c2):
                iv = idxall[pl.ds(j * 16, 16)]
                outb[pl.ds(j * 16, 16)] = plsc.load_gather(rowb, [iv])
                return c2

            lax.fori_loop(0, kp // 16, inner, 0)
            pltpu.sync_copy(outb, out_hbm.at[base + r])
            return c

        lax.fori_loop(0, rpw, row, 0)

    return k(T, idx)


_CHUNK = 819200
_EPT3 = _E // _NW
_EPT3P = 10240


def _sc_build_bcd(src, dst, ip):
    nwin = _EPT3P // 128

    @functools.partial(
        pl.kernel,
        out_type=(jax.ShapeDtypeStruct((_NP * _K1P,), jnp.float32),
                  jax.ShapeDtypeStruct((_K1P * _NP,), jnp.float32),
                  jax.ShapeDtypeStruct((_K1P * _K1P,), jnp.float32)),
        scratch_types=[
            pltpu.VMEM((_EPT3P,), jnp.int32),
            pltpu.VMEM((_EPT3P,), jnp.int32),
            pltpu.VMEM((_EPT3P,), jnp.int32),
            pltpu.VMEM((_EPT3P,), jnp.int32),
            pltpu.VMEM((_NP,), jnp.int32),
            pltpu.VMEM((10240,), jnp.float32),
            pltpu.VMEM((128,), jnp.int32),
            pltpu.VMEM((128,), jnp.int32),
            pltpu.VMEM((128,), jnp.float32),
            pltpu.VMEM_SHARED((_CHUNK + 16,), jnp.float32),
            pltpu.SemaphoreType.DMA,
            pltpu.SemaphoreType.DMA,
        ],
        **_SC_MESH,
    )
    def k(src_hbm, dst_hbm, ip_hbm, b_hbm, c_hbm, d_hbm,
          srcc, dstc, ipsc, ipdc, iptab, zbuf, idxw, idxw1, ones_v, chunk,
          sem0, sem1):
        cid = lax.axis_index("c")
        sid = lax.axis_index("s")
        ebase = (cid * 16 + sid) * _EPT3
        pltpu.sync_copy(src_hbm.at[pl.ds(ebase, _EPT3)],
                        srcc.at[pl.ds(0, _EPT3)])
        pltpu.sync_copy(dst_hbm.at[pl.ds(ebase, _EPT3)],
                        dstc.at[pl.ds(0, _EPT3)])
        pltpu.sync_copy(ip_hbm, iptab)

        def fill(i, c):
            zbuf[pl.ds(i * 16, 16)] = jnp.zeros((16,), jnp.float32)
            return c

        lax.fori_loop(0, 10240 // 16, fill, 0)
        _fill_1d(ones_v, 128, 1.0, jnp.float32)

        def pre(i, c):
            gi = i * 16 + lax.iota(jnp.int32, 16)
            inb = gi < _EPT3
            s = jnp.where(inb, srcc[pl.ds(i * 16, 16)], 0)
            d = jnp.where(inb, dstc[pl.ds(i * 16, 16)], 0)
            ok = inb & (s != d)
            ipsc[pl.ds(i * 16, 16)] = jnp.where(
                ok, plsc.load_gather(iptab, [s]), -1)
            ipdc[pl.ds(i * 16, 16)] = jnp.where(
                ok, plsc.load_gather(iptab, [d]), -1)
            return c

        lax.fori_loop(0, _EPT3P // 16, pre, 0)

        tel = sid * 51200

        def phase(rowarr, colarr, out_hbm, rows, width, nchunks):
            def one_chunk(j, c):
                r0 = (cid * nchunks + j) * rows

                def zro(z, c2):
                    pltpu.sync_copy(
                        zbuf, chunk.at[pl.ds(tel + z * 10240, 10240)])
                    return c2

                lax.fori_loop(0, 5, zro, 0)
                plsc.subcore_barrier()

                dump = _CHUNK + lax.iota(jnp.int32, 16)

                def build(w, buf):
                    for v in range(8):
                        sl = pl.ds(w * 128 + v * 16, 16)
                        rv = rowarr[sl]
                        cv = colarr[sl]
                        ok = (rv >= r0) & (rv < r0 + rows) & (cv >= 0)
                        lidx = (rv - r0) * width + cv
                        buf[pl.ds(v * 16, 16)] = jnp.where(ok, lidx, dump)

                def win(w, c2):
                    build(2 * w, idxw)
                    cp0 = pltpu.async_copy(ones_v, chunk.at[idxw], sem0,
                                           add=True)
                    build(2 * w + 1, idxw1)
                    cp1 = pltpu.async_copy(ones_v, chunk.at[idxw1], sem1,
                                           add=True)
                    cp0.wait()
                    cp1.wait()
                    return c2

                lax.fori_loop(0, nwin // 2, win, 0)
                plsc.subcore_barrier()
                pltpu.sync_copy(
                    chunk.at[pl.ds(tel, 51200)],
                    out_hbm.at[pl.ds(r0 * width + tel, 51200)])
                plsc.subcore_barrier()
                return c

            lax.fori_loop(0, nchunks, one_chunk, 0)

        phase(srcc, ipdc, b_hbm, 160, _K1P, 32)
        phase(ipsc, dstc, c_hbm, 80, _NP, 32)
        phase(ipsc, ipdc, d_hbm, 160, _K1P, 16)

    return k(src, dst, ip)


def _ew_gcn0_body(p0_ref, p1_ref, y_ref, d_ref, b_ref, o_ref, *, act):
    r = (p0_ref[...] + p1_ref[...] + 2.0 * y_ref[...]) * d_ref[...] + b_ref[...]
    if act:
        r = jnp.where(r > 0, r, jnp.exp(r) - 1.0)
    o_ref[...] = r


def _ew_gcn0(P0, P1, y, dinv_col, b, act, bm=512):
    n = y.shape[0]
    return pl.pallas_call(
        functools.partial(_ew_gcn0_body, act=act),
        grid=(n // bm,),
        in_specs=[pl.BlockSpec((bm, _H), lambda i: (i, 0))] * 3
        + [pl.BlockSpec((bm, 1), lambda i: (i, 0)),
           pl.BlockSpec((1, _H), lambda i: (0, 0))],
        out_specs=pl.BlockSpec((bm, _H), lambda i: (i, 0)),
        out_shape=jax.ShapeDtypeStruct((n, _H), jnp.float32),
    )(P0, P1, y, dinv_col, b.reshape(1, _H))


def _mask_add_body(res_ref, g_ref, ip_ref, o_ref):
    o_ref[...] = res_ref[...] + jnp.where(ip_ref[...] >= 0, g_ref[...], 0.0)


def _mask_add(res, g, ip_col, bm=None):
    n = res.shape[0]
    bm = bm or _bs(n)
    return pl.pallas_call(
        _mask_add_body,
        grid=(n // bm,),
        in_specs=[pl.BlockSpec((bm, _H), lambda i: (i, 0)),
                  pl.BlockSpec((bm, _H), lambda i: (i, 0)),
                  pl.BlockSpec((bm, 1), lambda i: (i, 0))],
        out_specs=pl.BlockSpec((bm, _H), lambda i: (i, 0)),
        out_shape=jax.ShapeDtypeStruct((n, _H), jnp.float32),
    )(res, g, ip_col)


def _adj_comb_body(a_ref, d_ref, o_ref, *, bm, bn):
    i = pl.program_id(0)
    j = pl.program_id(1)
    r = a_ref[...] + 2.0 * d_ref[...]
    rows = i * bm + jax.lax.broadcasted_iota(jnp.int32, (bm, bn), 0)
    cols = j * bn + jax.lax.broadcasted_iota(jnp.int32, (bm, bn), 1)
    o_ref[...] = jnp.where(rows == cols, 0.0, r)


def _adj_combine(A, D, bm=None, bn=None):
    m, n = A.shape
    bm = bm or _bs(m)
    bn = bn or _bs(n)
    return pl.pallas_call(
        functools.partial(_adj_comb_body, bm=bm, bn=bn),
        grid=(m // bm, n // bn),
        in_specs=[pl.BlockSpec((bm, bn), lambda i, j: (i, j)),
                  pl.BlockSpec((bm, bn), lambda i, j: (i, j))],
        out_specs=pl.BlockSpec((bm, bn), lambda i, j: (i, j)),
        out_shape=jax.ShapeDtypeStruct((m, n), jnp.float32),
    )(A, D)


def _topk_stage(h, p, n_real, k, kp):
    s = _scores(h, p)
    np_ = h.shape[0]
    ip2d = _topk_ip(s.reshape(-1, _H), n_real, k)
    ip = ip2d.reshape(-1)
    perm, vals = _sc_compact(ip, s.reshape(-1), np_, kp, zero_row=n_real)
    return perm, vals, ip


def _restricted_square(M, perm):
    C = _sc_rowgather(M, perm)
    D = _sc_colgather(C, perm)
    G = _big_mm(C, M)
    Gc = _sc_colgather(G, perm)
    return _adj_combine(Gc, D)


def _gcn_dense(M, x, W, b, act, extra=None):
    deg = _colsum(M) + 2.0
    dinv_col = jax.lax.rsqrt(deg).reshape(-1, 1)
    scale = dinv_col if extra is None else dinv_col * extra
    y = _feat_mm(x, W, jnp.zeros((_H,), jnp.float32), scale=scale)
    return _gcn_agg(M, y, dinv_col, b, act)


def kernel(x, edge_index, batch, Wd0, bd0, Wd1, bd1, Wd2, bd2, Wd3, bd3,
           p1, p2, p3, Wu0, bu0, Wu1, bu1, Wu2, bu2, Wo, bo):
    src = edge_index[0]
    dst = edge_index[1]
    xp = jnp.pad(x, ((0, _NP - _N), (0, 0)))

    degp = _sc_deg(dst)
    deg0 = degp[0] + degp[1] + 2.0
    dinv0c = jax.lax.rsqrt(deg0).reshape(-1, 1)

    def gcn_edges(xin, W, b, act):
        y = _feat_mm(xin, W, jnp.zeros((_H,), jnp.float32), scale=dinv0c)
        P = _sc_agg(y, src, dst)
        return _ew_gcn0(P[0], P[1], y, dinv0c, b, act)

    h0 = gcn_edges(xp, Wd0, bd0, True)

    perm1, vals1, ip1 = _topk_stage(h0, p1, _N, _K1, _K1P)
    Bf, Cf, Df = _sc_build_bcd(src, dst, ip1)
    M1 = _big_mm(Cf.reshape(_K1P, _NP), Bf.reshape(_NP, _K1P),
                 D=Df.reshape(_K1P, _K1P), diag_zero=True)
    h1 = _gcn_dense(M1, _sc_rowgather(h0, perm1), Wd1, bd1, True,
                    extra=vals1.reshape(-1, 1))

    perm2, vals2, ip2 = _topk_stage(h1, p2, _K1, _K2, _K2P)
    M2 = _restricted_square(M1, perm2)
    h2 = _gcn_dense(M2, _sc_rowgather(h1, perm2), Wd2, bd2, True,
                    extra=vals2.reshape(-1, 1))

    perm3, vals3, ip3 = _topk_stage(h2, p3, _K2, _K3, _K3P)
    M3 = _restricted_square(M2, perm3)
    h3 = _gcn_dense(M3, _sc_rowgather(h2, perm3), Wd3, bd3, True,
                    extra=vals3.reshape(-1, 1))

    u = _mask_add(h2, _sc_rowgather(h3, ip3, clamp=True), ip3.reshape(-1, 1))
    u = _gcn_dense(M2, u, Wu0, bu0, True)
    u = _mask_add(h1, _sc_rowgather(u, ip2, clamp=True), ip2.reshape(-1, 1))
    u = _gcn_dense(M1, u, Wu1, bu1, True)
    u = _mask_add(h0, _sc_rowgather(u, ip1, clamp=True), ip1.reshape(-1, 1))
    u = gcn_edges(u, Wu2, bu2, False)

    return _readout(u, Wo, bo, _N)


# speedup vs baseline: 2.2514x; 1.0296x over previous
import functools
import math

import jax
import jax.numpy as jnp
from jax import lax
from jax.experimental import pallas as pl
from jax.experimental.pallas import tpu as pltpu
from jax.experimental.pallas import tpu_sc as plsc

_SC_MESH = dict(mesh=plsc.VectorSubcoreMesh(core_axis_name="c",
                                            subcore_axis_name="s"),
                compiler_params=pltpu.CompilerParams(
                    needs_layout_passes=False))
_NW = 32

_N = 10000
_E = 320000
_H = 128
_NP = 10240
_K1, _K1P = 5000, 5120
_K2, _K2P = 2500, 2560
_K3, _K3P = 1250, 1280

_INT_MIN = -2147483648


def _bs(n):
    return 512 if n % 512 == 0 else 256


def _bsl(n):
    for b in (1280, 1024, 512):
        if n % b == 0:
            return b
    return 256


def _feat_mm_body(x_ref, w_ref, s_ref, b_ref, o_ref, *, act, use_scale):
    acc = jnp.dot(x_ref[...], w_ref[...], preferred_element_type=jnp.float32)
    if use_scale:
        acc = acc * s_ref[...]
    acc = acc + b_ref[...]
    if act:
        acc = jnp.where(acc > 0, acc, jnp.exp(acc) - 1.0)
    o_ref[...] = acc


def _feat_mm(x, W, b, scale=None, act=False, bm=None):
    n = x.shape[0]
    bm = bm or _bs(n)
    use_scale = scale is not None
    if scale is None:
        scale = jnp.zeros((n, 1), jnp.float32)
    grid = (n // bm,)
    return pl.pallas_call(
        functools.partial(_feat_mm_body, act=act, use_scale=use_scale),
        grid=grid,
        in_specs=[
            pl.BlockSpec((bm, _H), lambda i: (i, 0)),
            pl.BlockSpec((_H, _H), lambda i: (0, 0)),
            pl.BlockSpec((bm, 1), lambda i: (i, 0)),
            pl.BlockSpec((1, _H), lambda i: (0, 0)),
        ],
        out_specs=pl.BlockSpec((bm, _H), lambda i: (i, 0)),
        out_shape=jax.ShapeDtypeStruct((n, _H), jnp.float32),
    )(x, W, scale, b.reshape(1, _H))


def _colsum_body(m_ref, o_ref):
    i = pl.program_id(1)

    @pl.when(i == 0)
    def _():
        o_ref[...] = jnp.zeros_like(o_ref)

    o_ref[...] += jnp.sum(m_ref[...], axis=0, keepdims=True)


def _colsum(M, bi=None, bj=None):
    n, m = M.shape
    bi = bi or _bs(n)
    bj = bj or _bs(m)
    return pl.pallas_call(
        _colsum_body,
        grid=(m // bj, n // bi),
        in_specs=[pl.BlockSpec((bi, bj), lambda j, i: (i, j))],
        out_specs=pl.BlockSpec((1, bj), lambda j, i: (0, j)),
        out_shape=jax.ShapeDtypeStruct((1, m), jnp.float32),
    )(M)


def _agg_body(m_ref, y_ref, y2_ref, d_ref, b_ref, o_ref, acc, *, nk, act):
    k = pl.program_id(1)

    @pl.when(k == 0)
    def _():
        acc[...] = jnp.zeros_like(acc)

    acc[...] += jax.lax.dot_general(
        m_ref[...], y_ref[...], (((0,), (0,)), ((), ())),
        preferred_element_type=jnp.float32)

    @pl.when(k == nk - 1)
    def _():
        r = (acc[...] + 2.0 * y2_ref[...]) * d_ref[...] + b_ref[...]
        if act:
            r = jnp.where(r > 0, r, jnp.exp(r) - 1.0)
        o_ref[...] = r


def _gcn_agg(M, y, dinv_col, b, act, bi=None, bk=None):
    n = M.shape[0]
    bi = bi or _bs(n)
    bk = bk or _bs(n)
    return pl.pallas_call(
        functools.partial(_agg_body, nk=n // bk, act=act),
        grid=(n // bi, n // bk),
        in_specs=[
            pl.BlockSpec((bk, bi), lambda i, k: (k, i)),
            pl.BlockSpec((bk, _H), lambda i, k: (k, 0)),
            pl.BlockSpec((bi, _H), lambda i, k: (i, 0)),
            pl.BlockSpec((bi, 1), lambda i, k: (i, 0)),
            pl.BlockSpec((1, _H), lambda i, k: (0, 0)),
        ],
        out_specs=pl.BlockSpec((bi, _H), lambda i, k: (i, 0)),
        out_shape=jax.ShapeDtypeStruct((n, _H), jnp.float32),
        scratch_shapes=[pltpu.VMEM((bi, _H), jnp.float32)],
    )(M, y, y, dinv_col, b.reshape(1, _H))


def _gcn_dense(M, x, W, b, act):
    deg = _colsum(M) + 2.0
    dinv_col = jax.lax.rsqrt(deg).reshape(-1, 1)
    y = _feat_mm(x, W, jnp.zeros((_H,), jnp.float32), scale=dinv_col)
    return _gcn_agg(M, y, dinv_col, b, act)


def _bigmm_body(p_ref, q_ref, d_ref, o_ref, acc, *, nk, use_d, diag_zero, bm, bn):
    mi = pl.program_id(0)
    nj = pl.program_id(1)
    k = pl.program_id(2)

    @pl.when(k == 0)
    def _():
        acc[...] = jnp.zeros_like(acc)

    acc[...] += jnp.dot(p_ref[...].astype(jnp.bfloat16),
                        q_ref[...].astype(jnp.bfloat16),
                        preferred_element_type=jnp.float32)

    @pl.when(k == nk - 1)
    def _():
        r = acc[...]
        if use_d:
            r = r + 2.0 * d_ref[...]
        if diag_zero:
            rows = mi * bm + jax.lax.broadcasted_iota(jnp.int32, (bm, bn), 0)
            cols = nj * bn + jax.lax.broadcasted_iota(jnp.int32, (bm, bn), 1)
            r = jnp.where(rows == cols, 0.0, r)
        o_ref[...] = r


def _big_mm(P, Q, D=None, diag_zero=False, bm=None, bn=None, bk=None):
    m, kdim = P.shape
    n = Q.shape[1]
    bm = bm or _bsl(m)
    bn = bn or _bsl(n)
    bk = bk or _bs(kdim)
    use_d = D is not None
    if D is None:
        D = jnp.zeros((bm, bn), jnp.float32)
        d_spec = pl.BlockSpec((bm, bn), lambda i, j, k: (0, 0))
    else:
        d_spec = pl.BlockSpec((bm, bn), lambda i, j, k: (i, j))
    return pl.pallas_call(
        functools.partial(_bigmm_body, nk=kdim // bk, use_d=use_d,
                          diag_zero=diag_zero, bm=bm, bn=bn),
        grid=(m // bm, n // bn, kdim // bk),
        in_specs=[
            pl.BlockSpec((bm, bk), lambda i, j, k: (i, k)),
            pl.BlockSpec((bk, bn), lambda i, j, k: (k, j)),
            d_spec,
        ],
        out_specs=pl.BlockSpec((bm, bn), lambda i, j, k: (i, j)),
        out_shape=jax.ShapeDtypeStruct((m, n), jnp.float32),
        scratch_shapes=[pltpu.VMEM((bm, bn), jnp.float32)],
    )(P, Q, D)


def _score_body(p_ref, h_ref, o_ref):
    pr = p_ref[...]
    inv_norm = jax.lax.rsqrt(jnp.sum(pr * pr))
    s = jax.lax.dot_general(pr, h_ref[...], (((1,), (1,)), ((), ())),
                            preferred_element_type=jnp.float32)
    o_ref[...] = jnp.tanh(s * inv_norm)


def _scores(h, p):
    n = h.shape[0]
    nb = n // _H
    return pl.pallas_call(
        _score_body,
        grid=(nb,),
        in_specs=[
            pl.BlockSpec((1, _H), lambda b: (0, 0)),
            pl.BlockSpec((_H, _H), lambda b: (b, 0)),
        ],
        out_specs=pl.BlockSpec((1, _H), lambda b: (0, b)),
        out_shape=jax.ShapeDtypeStruct((1, n), jnp.float32),
    )(p.reshape(1, _H), h)


def _topk_body(s_ref, ip_ref, *, n_real, k):
    nb = s_ref.shape[0]
    s = s_ref[...]
    rows = jax.lax.broadcasted_iota(jnp.int32, (nb, _H), 0)
    cols = jax.lax.broadcasted_iota(jnp.int32, (nb, _H), 1)
    gidx = rows * _H + cols
    valid = gidx < n_real
    u = jax.lax.bitcast_convert_type(s, jnp.int32)
    key = u ^ jnp.where(u < 0, 0x7FFFFFFF, 0)
    key = jnp.where(valid, key, _INT_MIN)

    def body(b, prefix):
        cand = jnp.where(b == 31, jnp.int32(0), prefix + (jnp.int32(1) << b))
        cnt = jnp.sum(jnp.where(key >= cand, 1.0, 0.0))
        return jnp.where(cnt >= k, cand, prefix)

    thr = jax.lax.fori_loop(0, 32, lambda i, c: body(31 - i, c),
                            jnp.int32(_INT_MIN), unroll=True)

    gt = jnp.where(key > thr, 1.0, 0.0)
    eq = jnp.where(key == thr, 1.0, 0.0)
    cnt_gt = jnp.sum(gt)
    r_need = k - cnt_gt

    upper = jnp.where(
        jax.lax.broadcasted_iota(jnp.int32, (_H, _H), 0)
        < jax.lax.broadcasted_iota(jnp.int32, (_H, _H), 1), 1.0, 0.0)
    lower_nb = jnp.where(
        jax.lax.broadcasted_iota(jnp.int32, (nb, nb), 1)
        < jax.lax.broadcasted_iota(jnp.int32, (nb, nb), 0), 1.0, 0.0)
    ones_col = jnp.ones((_H, 1), jnp.float32)

    def ex_prefix(m):
        within = jnp.dot(m, upper, preferred_element_type=jnp.float32)
        rowsum = jnp.dot(m, ones_col, preferred_element_type=jnp.float32)
        rowoff = jnp.dot(lower_nb, rowsum, preferred_element_type=jnp.float32)
        return within + rowoff

    eq_rank = ex_prefix(eq)
    sel = (gt > 0) | ((eq > 0) & (eq_rank < r_need))
    self32 = jnp.where(sel, 1.0, 0.0)
    slot = ex_prefix(self32)
    ip_ref[...] = jnp.where(sel, slot.astype(jnp.int32), -1)


def _topk_ip(s2d, n_real, k):
    nb = s2d.shape[0]
    return pl.pallas_call(
        functools.partial(_topk_body, n_real=n_real, k=k),
        in_specs=[pl.BlockSpec((nb, _H), lambda: (0, 0))],
        out_specs=pl.BlockSpec((nb, _H), lambda: (0, 0)),
        out_shape=jax.ShapeDtypeStruct((nb, _H), jnp.int32),
    )(s2d)


def _readout_body(u_ref, w_ref, b_ref, o_ref, acc, *, n_real, nb, bm):
    i = pl.program_id(0)

    @pl.when(i == 0)
    def _():
        acc[...] = jnp.zeros_like(acc)

    rows = i * bm + jax.lax.broadcasted_iota(jnp.int32, (bm, _H), 0)
    blk = jnp.where(rows < n_real, u_ref[...], 0.0)
    acc[...] += jnp.sum(blk, axis=0, keepdims=True)

    @pl.when(i == nb - 1)
    def _():
        g = acc[...] * (1.0 / n_real)
        o_ref[...] = jnp.dot(g, w_ref[...],
                             preferred_element_type=jnp.float32) + b_ref[...]


def _readout(u, Wo, bo, n_real, bm=512):
    n = u.shape[0]
    nb = n // bm
    return pl.pallas_call(
        functools.partial(_readout_body, n_real=n_real, nb=nb, bm=bm),
        grid=(nb,),
        in_specs=[
            pl.BlockSpec((bm, _H), lambda i: (i, 0)),
            pl.BlockSpec((_H, _H), lambda i: (0, 0)),
            pl.BlockSpec((1, _H), lambda i: (0, 0)),
        ],
        out_specs=pl.BlockSpec((1, _H), lambda i: (0, 0)),
        out_shape=jax.ShapeDtypeStruct((1, _H), jnp.float32),
        scratch_shapes=[pltpu.VMEM((1, _H), jnp.float32)],
    )(u, Wo, bo.reshape(1, _H))


def _sc_wid():
    return lax.axis_index("c") * 16 + lax.axis_index("s")


def _fill_1d(ref, n, val, dtype):
    for i in range(n // 16):
        ref[pl.ds(i * 16, 16)] = jnp.full((16,), val, dtype)


def _sc_deg(dst):
    ept = _E // _NW
    nwin = ept // 80
    rows = _NP // 16

    @functools.partial(
        pl.kernel,
        out_type=jax.ShapeDtypeStruct((2, _NP), jnp.float32),
        scratch_types=[
            pltpu.VMEM((80,), jnp.int32),
            pltpu.VMEM((80,), jnp.float32),
            pltpu.VMEM((rows,), jnp.float32),
            pltpu.VMEM_SHARED((_NP,), jnp.float32),
        ],
        **_SC_MESH,
    )
    def k(dst_hbm, out_hbm, idxw, ones_v, zbuf, degsh):
        cid = lax.axis_index("c")
        sid = lax.axis_index("s")
        _fill_1d(ones_v, 80, 1.0, jnp.float32)
        _fill_1d(zbuf, rows, 0.0, jnp.float32)
        pltpu.sync_copy(zbuf, degsh.at[pl.ds(sid * rows, rows)])
        plsc.subcore_barrier()
        base = (cid * 16 + sid) * ept

        def win(w, c):
            pltpu.sync_copy(dst_hbm.at[pl.ds(base + w * 80, 80)], idxw)
            pltpu.sync_copy(ones_v, degsh.at[idxw], add=True)
            return c

        lax.fori_loop(0, nwin, win, 0)
        plsc.subcore_barrier()
        pltpu.sync_copy(degsh.at[pl.ds(sid * rows, rows)],
                        out_hbm.at[cid, pl.ds(sid * rows, rows)])

    return k(dst)


def _sc_agg(y, src, dst):
    ept = _E // _NW
    nwin = ept // 80
    rows = _NP // 16

    @functools.partial(
        pl.kernel,
        out_type=jax.ShapeDtypeStruct((2, _NP, _H), jnp.float32),
        scratch_types=[
            pltpu.VMEM((80,), jnp.int32),
            pltpu.VMEM((80,), jnp.int32),
            pltpu.VMEM((80,), jnp.int32),
            pltpu.VMEM((80,), jnp.int32),
            pltpu.VMEM((80, _H), jnp.float32),
            pltpu.VMEM((80, _H), jnp.float32),
            pltpu.VMEM((16, _H), jnp.float32),
            pltpu.VMEM_SHARED((_NP, _H), jnp.float32),
            pltpu.SemaphoreType.DMA,
            pltpu.SemaphoreType.DMA,
            pltpu.SemaphoreType.DMA,
            pltpu.SemaphoreType.DMA,
        ],
        **_SC_MESH,
    )
    def k(y_hbm, src_hbm, dst_hbm, out_hbm, sidx, didx, sidx1, didx1,
          rbuf, rbuf1, zbuf, accsh, sem0, sem1, sem2, sem3):
        cid = lax.axis_index("c")
        sid = lax.axis_index("s")
        for r in range(16):
            _fill_1d(zbuf.at[r], _H, 0.0, jnp.float32)

        def zro(r, c):
            pltpu.sync_copy(zbuf, accsh.at[pl.ds(sid * rows + r * 16, 16)])
            return c

        lax.fori_loop(0, rows // 16, zro, 0)
        plsc.subcore_barrier()
        base = (cid * 16 + sid) * ept

        def win(w, c):
            pltpu.sync_copy(src_hbm.at[pl.ds(base + 2 * w * 80, 80)], sidx)
            pltpu.sync_copy(dst_hbm.at[pl.ds(base + 2 * w * 80, 80)], didx)
            g0 = pltpu.async_copy(y_hbm.at[sidx], rbuf, sem0)
            pltpu.sync_copy(src_hbm.at[pl.ds(base + (2 * w + 1) * 80, 80)],
                            sidx1)
            pltpu.sync_copy(dst_hbm.at[pl.ds(base + (2 * w + 1) * 80, 80)],
                            didx1)
            g1 = pltpu.async_copy(y_hbm.at[sidx1], rbuf1, sem1)
            g0.wait()
            s0 = pltpu.async_copy(rbuf, accsh.at[didx], sem2, add=True)
            g1.wait()
            s1 = pltpu.async_copy(rbuf1, accsh.at[didx1], sem3, add=True)
            s0.wait()
            s1.wait()
            return c

        lax.fori_loop(0, nwin // 2, win, 0)
        pltpu.sync_copy(src_hbm.at[pl.ds(base + (nwin - 1) * 80, 80)], sidx)
        pltpu.sync_copy(dst_hbm.at[pl.ds(base + (nwin - 1) * 80, 80)], didx)
        pltpu.async_copy(y_hbm.at[sidx], rbuf, sem0).wait()
        pltpu.sync_copy(rbuf, accsh.at[didx], add=True)
        plsc.subcore_barrier()
        pltpu.sync_copy(accsh.at[pl.ds(sid * rows, rows)],
                        out_hbm.at[cid, pl.ds(sid * rows, rows)])

    return k(y, src, dst)


def _sc_compact(ip_flat, s_flat, n, kp, zero_row):
    nwin = n // 2048

    @functools.partial(
        pl.kernel,
        out_type=(jax.ShapeDtypeStruct((kp,), jnp.int32),
                  jax.ShapeDtypeStruct((kp,), jnp.float32)),
        scratch_types=[
            pltpu.VMEM((2048,), jnp.int32),
            pltpu.VMEM((2048,), jnp.float32),
            pltpu.VMEM((kp,), jnp.int32),
            pltpu.VMEM((kp,), jnp.float32),
        ],
        **_SC_MESH,
    )
    def k(ip_hbm, s_hbm, perm_hbm, vals_hbm, ipw, sw, permb, valsb):
        wid = _sc_wid()

        @pl.when(wid == 0)
        def _():
            _fill_1d(permb, kp, zero_row, jnp.int32)
            _fill_1d(valsb, kp, 0.0, jnp.float32)

            def win(w, c):
                pltpu.sync_copy(ip_hbm.at[pl.ds(w * 2048, 2048)], ipw)
                pltpu.sync_copy(s_hbm.at[pl.ds(w * 2048, 2048)], sw)

                def inner(i, c2):
                    idx = ipw[pl.ds(i * 16, 16)]
                    msk = idx >= 0
                    idx2 = jnp.maximum(idx, 0)
                    gi = (w * 2048 + i * 16
                          + lax.iota(jnp.int32, 16))
                    plsc.store_scatter(permb, [idx2], gi, mask=msk)
                    plsc.store_scatter(valsb, [idx2], sw[pl.ds(i * 16, 16)],
                                       mask=msk)
                    return c2

                lax.fori_loop(0, 128, inner, 0)
                return c

            lax.fori_loop(0, nwin, win, 0)
            pltpu.sync_copy(permb, perm_hbm)
            pltpu.sync_copy(valsb, vals_hbm)

    return k(ip_flat, s_flat)


def _sc_rowgather(T, idx, clamp=False):
    kp = idx.shape[0]
    w = T.shape[1]
    rpw = kp // _NW
    nwin = rpw // 16

    @functools.partial(
        pl.kernel,
        out_type=jax.ShapeDtypeStruct((kp, w), T.dtype),
        scratch_types=[
            pltpu.VMEM((16,), jnp.int32),
            pltpu.VMEM((16, w), T.dtype),
            pltpu.SemaphoreType.DMA,
        ],
        **_SC_MESH,
    )
    def k(t_hbm, idx_hbm, out_hbm, idxw, rbuf, sem):
        base = _sc_wid() * rpw

        def win(r, c):
            pltpu.sync_copy(idx_hbm.at[pl.ds(base + r * 16, 16)], idxw)
            if clamp:
                idxw[pl.ds(0, 16)] = jnp.maximum(idxw[pl.ds(0, 16)], 0)
            pltpu.async_copy(t_hbm.at[idxw], rbuf, sem).wait()
            pltpu.sync_copy(rbuf, out_hbm.at[pl.ds(base + r * 16, 16)])
            return c

        lax.fori_loop(0, nwin, win, 0)

    return k(T, idx)


def _sc_colgather(T, idx):
    m, w = T.shape
    kp = idx.shape[0]
    rpw = m // _NW

    @functools.partial(
        pl.kernel,
        out_type=jax.ShapeDtypeStruct((m, kp), jnp.float32),
        scratch_types=[
            pltpu.VMEM((kp,), jnp.int32),
            pltpu.VMEM((w,), jnp.float32),
            pltpu.VMEM((kp,), jnp.float32),
        ],
        **_SC_MESH,
    )
    def k(t_hbm, idx_hbm, out_hbm, idxall, rowb, outb):
        base = _sc_wid() * rpw
        pltpu.sync_copy(idx_hbm, idxall)

        def row(r, c):
            pltpu.sync_copy(t_hbm.at[base + r], rowb)

            def inner(j, c2):
                iv = idxall[pl.ds(j * 16, 16)]
                outb[pl.ds(j * 16, 16)] = plsc.load_gather(rowb, [iv])
                return c2

            lax.fori_loop(0, kp // 16, inner, 0)
            pltpu.sync_copy(outb, out_hbm.at[base + r])
            return c

        lax.fori_loop(0, rpw, row, 0)

    return k(T, idx)


_CHUNK = 819200
_EPT3 = _E // _NW
_EPT3P = 10240


def _sc_build_bcd(src, dst, ip):
    nwin = _EPT3P // 128

    @functools.partial(
        pl.kernel,
        out_type=(jax.ShapeDtypeStruct((_NP * _K1P,), jnp.float32),
                  jax.ShapeDtypeStruct((_K1P * _NP,), jnp.float32),
                  jax.ShapeDtypeStruct((_K1P * _K1P,), jnp.float32)),
        scratch_types=[
            pltpu.VMEM((_EPT3P,), jnp.int32),
            pltpu.VMEM((_EPT3P,), jnp.int32),
            pltpu.VMEM((_EPT3P,), jnp.int32),
            pltpu.VMEM((_EPT3P,), jnp.int32),
            pltpu.VMEM((_NP,), jnp.int32),
            pltpu.VMEM((10240,), jnp.float32),
            pltpu.VMEM((128,), jnp.int32),
            pltpu.VMEM((128,), jnp.int32),
            pltpu.VMEM((128,), jnp.float32),
            pltpu.VMEM_SHARED((_CHUNK + 16,), jnp.float32),
            pltpu.SemaphoreType.DMA,
            pltpu.SemaphoreType.DMA,
        ],
        **_SC_MESH,
    )
    def k(src_hbm, dst_hbm, ip_hbm, b_hbm, c_hbm, d_hbm,
          srcc, dstc, ipsc, ipdc, iptab, zbuf, idxw, idxw1, ones_v, chunk,
          sem0, sem1):
        cid = lax.axis_index("c")
        sid = lax.axis_index("s")
        ebase = (cid * 16 + sid) * _EPT3
        pltpu.sync_copy(src_hbm.at[pl.ds(ebase, _EPT3)],
                        srcc.at[pl.ds(0, _EPT3)])
        pltpu.sync_copy(dst_hbm.at[pl.ds(ebase, _EPT3)],
                        dstc.at[pl.ds(0, _EPT3)])
        pltpu.sync_copy(ip_hbm, iptab)

        def fill(i, c):
            zbuf[pl.ds(i * 16, 16)] = jnp.zeros((16,), jnp.float32)
            return c

        lax.fori_loop(0, 10240 // 16, fill, 0)
        _fill_1d(ones_v, 128, 1.0, jnp.float32)

        def pre(i, c):
            gi = i * 16 + lax.iota(jnp.int32, 16)
            inb = gi < _EPT3
            s = jnp.where(inb, srcc[pl.ds(i * 16, 16)], 0)
            d = jnp.where(inb, dstc[pl.ds(i * 16, 16)], 0)
            ok = inb & (s != d)
            ipsc[pl.ds(i * 16, 16)] = jnp.where(
                ok, plsc.load_gather(iptab, [s]), -1)
            ipdc[pl.ds(i * 16, 16)] = jnp.where(
                ok, plsc.load_gather(iptab, [d]), -1)
            return c

        lax.fori_loop(0, _EPT3P // 16, pre, 0)

        tel = sid * 51200

        def phase(rowarr, colarr, out_hbm, rows, width, nchunks):
            def one_chunk(j, c):
                r0 = (cid * nchunks + j) * rows

                def zro(z, c2):
                    pltpu.sync_copy(
                        zbuf, chunk.at[pl.ds(tel + z * 10240, 10240)])
                    return c2

                lax.fori_loop(0, 5, zro, 0)
                plsc.subcore_barrier()

                dump = _CHUNK + lax.iota(jnp.int32, 16)

                def build(w, buf):
                    for v in range(8):
                        sl = pl.ds(w * 128 + v * 16, 16)
                        rv = rowarr[sl]
                        cv = colarr[sl]
                        ok = (rv >= r0) & (rv < r0 + rows) & (cv >= 0)
                        lidx = (rv - r0) * width + cv
                        buf[pl.ds(v * 16, 16)] = jnp.where(ok, lidx, dump)

                def win(w, c2):
                    build(2 * w, idxw)
                    cp0 = pltpu.async_copy(ones_v, chunk.at[idxw], sem0,
                                           add=True)
                    build(2 * w + 1, idxw1)
                    cp1 = pltpu.async_copy(ones_v, chunk.at[idxw1], sem1,
                                           add=True)
                    cp0.wait()
                    cp1.wait()
                    return c2

                lax.fori_loop(0, nwin // 2, win, 0)
                plsc.subcore_barrier()
                pltpu.sync_copy(
                    chunk.at[pl.ds(tel, 51200)],
                    out_hbm.at[pl.ds(r0 * width + tel, 51200)])
                plsc.subcore_barrier()
                return c

            lax.fori_loop(0, nchunks, one_chunk, 0)

        phase(srcc, ipdc, b_hbm, 160, _K1P, 32)
        phase(ipsc, dstc, c_hbm, 80, _NP, 32)
        phase(ipsc, ipdc, d_hbm, 160, _K1P, 16)

    return k(src, dst, ip)


def _ew_gcn0_body(p0_ref, p1_ref, y_ref, d_ref, b_ref, o_ref, *, act):
    r = (p0_ref[...] + p1_ref[...] + 2.0 * y_ref[...]) * d_ref[...] + b_ref[...]
    if act:
        r = jnp.where(r > 0, r, jnp.exp(r) - 1.0)
    o_ref[...] = r


def _ew_gcn0(P0, P1, y, dinv_col, b, act, bm=512):
    n = y.shape[0]
    return pl.pallas_call(
        functools.partial(_ew_gcn0_body, act=act),
        grid=(n // bm,),
        in_specs=[pl.BlockSpec((bm, _H), lambda i: (i, 0))] * 3
        + [pl.BlockSpec((bm, 1), lambda i: (i, 0)),
           pl.BlockSpec((1, _H), lambda i: (0, 0))],
        out_specs=pl.BlockSpec((bm, _H), lambda i: (i, 0)),
        out_shape=jax.ShapeDtypeStruct((n, _H), jnp.float32),
    )(P0, P1, y, dinv_col, b.reshape(1, _H))


def _mask_add_body(res_ref, g_ref, ip_ref, o_ref):
    o_ref[...] = res_ref[...] + jnp.where(ip_ref[...] >= 0, g_ref[...], 0.0)


def _mask_add(res, g, ip_col, bm=None):
    n = res.shape[0]
    bm = bm or _bs(n)
    return pl.pallas_call(
        _mask_add_body,
        grid=(n // bm,),
        in_specs=[pl.BlockSpec((bm, _H), lambda i: (i, 0)),
                  pl.BlockSpec((bm, _H), lambda i: (i, 0)),
                  pl.BlockSpec((bm, 1), lambda i: (i, 0))],
        out_specs=pl.BlockSpec((bm, _H), lambda i: (i, 0)),
        out_shape=jax.ShapeDtypeStruct((n, _H), jnp.float32),
    )(res, g, ip_col)


def _adj_comb_body(a_ref, d_ref, o_ref, *, bm, bn):
    i = pl.program_id(0)
    j = pl.program_id(1)
    r = a_ref[...] + 2.0 * d_ref[...]
    rows = i * bm + jax.lax.broadcasted_iota(jnp.int32, (bm, bn), 0)
    cols = j * bn + jax.lax.broadcasted_iota(jnp.int32, (bm, bn), 1)
    o_ref[...] = jnp.where(rows == cols, 0.0, r)


def _adj_combine(A, D, bm=None, bn=None):
    m, n = A.shape
    bm = bm or _bs(m)
    bn = bn or _bs(n)
    return pl.pallas_call(
        functools.partial(_adj_comb_body, bm=bm, bn=bn),
        grid=(m // bm, n // bn),
        in_specs=[pl.BlockSpec((bm, bn), lambda i, j: (i, j)),
                  pl.BlockSpec((bm, bn), lambda i, j: (i, j))],
        out_specs=pl.BlockSpec((bm, bn), lambda i, j: (i, j)),
        out_shape=jax.ShapeDtypeStruct((m, n), jnp.float32),
    )(A, D)


def _topk_stage(h, p, n_real, k, kp):
    s = _scores(h, p)
    np_ = h.shape[0]
    ip2d = _topk_ip(s.reshape(-1, _H), n_real, k)
    ip = ip2d.reshape(-1)
    perm, vals = _sc_compact(ip, s.reshape(-1), np_, kp, zero_row=n_real)
    return perm, vals, ip


def _restricted_square(M, perm):
    C = _sc_rowgather(M, perm)
    D = _sc_colgather(C, perm)
    G = _big_mm(C, M)
    Gc = _sc_colgather(G, perm)
    return _adj_combine(Gc, D)


def _gcn_dense(M, x, W, b, act, extra=None):
    deg = _colsum(M) + 2.0
    dinv_col = jax.lax.rsqrt(deg).reshape(-1, 1)
    scale = dinv_col if extra is None else dinv_col * extra
    y = _feat_mm(x, W, jnp.zeros((_H,), jnp.float32), scale=scale)
    return _gcn_agg(M, y, dinv_col, b, act)


def kernel(x, edge_index, batch, Wd0, bd0, Wd1, bd1, Wd2, bd2, Wd3, bd3,
           p1, p2, p3, Wu0, bu0, Wu1, bu1, Wu2, bu2, Wo, bo):
    src = edge_index[0]
    dst = edge_index[1]
    xp = jnp.pad(x, ((0, _NP - _N), (0, 0)))

    degp = _sc_deg(dst)
    deg0 = degp[0] + degp[1] + 2.0
    dinv0c = jax.lax.rsqrt(deg0).reshape(-1, 1)

    def gcn_edges(xin, W, b, act):
        y = _feat_mm(xin, W, jnp.zeros((_H,), jnp.float32), scale=dinv0c)
        P = _sc_agg(y, src, dst)
        return _ew_gcn0(P[0], P[1], y, dinv0c, b, act)

    h0 = gcn_edges(xp, Wd0, bd0, True)

    perm1, vals1, ip1 = _topk_stage(h0, p1, _N, _K1, _K1P)
    Bf, Cf, Df = _sc_build_bcd(src, dst, ip1)
    M1 = _big_mm(Cf.reshape(_K1P, _NP), Bf.reshape(_NP, _K1P),
                 D=Df.reshape(_K1P, _K1P), diag_zero=True)
    h1 = _gcn_dense(M1, _sc_rowgather(h0, perm1), Wd1, bd1, True,
                    extra=vals1.reshape(-1, 1))

    perm2, vals2, ip2 = _topk_stage(h1, p2, _K1, _K2, _K2P)
    M2 = _restricted_square(M1, perm2)
    h2 = _gcn_dense(M2, _sc_rowgather(h1, perm2), Wd2, bd2, True,
                    extra=vals2.reshape(-1, 1))

    perm3, vals3, ip3 = _topk_stage(h2, p3, _K2, _K3, _K3P)
    M3 = _restricted_square(M2, perm3)
    h3 = _gcn_dense(M3, _sc_rowgather(h2, perm3), Wd3, bd3, True,
                    extra=vals3.reshape(-1, 1))

    u = _mask_add(h2, _sc_rowgather(h3, ip3, clamp=True), ip3.reshape(-1, 1))
    u = _gcn_dense(M2, u, Wu0, bu0, True)
    u = _mask_add(h1, _sc_rowgather(u, ip2, clamp=True), ip2.reshape(-1, 1))
    u = _gcn_dense(M1, u, Wu1, bu1, True)
    u = _mask_add(h0, _sc_rowgather(u, ip1, clamp=True), ip1.reshape(-1, 1))
    u = gcn_edges(u, Wu2, bu2, False)

    return _readout(u, Wo, bo, _N)
